# baseline (speedup 1.0000x reference)
import functools

import jax
import jax.numpy as jnp
from jax.experimental import pallas as pl
from jax.experimental.pallas import tpu as pltpu

N = 50000
E = 800000
B = 128
EMB = 128

NODE_BLK = 2000


def _gin_mlp_body(z_ref, w1_ref, b1_ref, w2_ref, b2_ref, o_ref):
    z = z_ref[...]
    z1 = jnp.maximum(jnp.dot(z, w1_ref[...], preferred_element_type=jnp.float32) + b1_ref[...], 0.0)
    o_ref[...] = jnp.dot(z1, w2_ref[...], preferred_element_type=jnp.float32) + b2_ref[...]


def _gin_mlp(z, w1, b1, w2, b2):
    return pl.pallas_call(
        _gin_mlp_body,
        grid=(N // NODE_BLK,),
        in_specs=[
            pl.BlockSpec((NODE_BLK, EMB), lambda i: (i, 0)),
            pl.BlockSpec((EMB, EMB), lambda i: (0, 0)),
            pl.BlockSpec((1, EMB), lambda i: (0, 0)),
            pl.BlockSpec((EMB, EMB), lambda i: (0, 0)),
            pl.BlockSpec((1, EMB), lambda i: (0, 0)),
        ],
        out_specs=pl.BlockSpec((NODE_BLK, EMB), lambda i: (i, 0)),
        out_shape=jax.ShapeDtypeStruct((N, EMB), jnp.float32),
    )(z, w1.reshape(EMB, EMB), b1.reshape(1, EMB), w2.reshape(EMB, EMB), b2.reshape(1, EMB))


def _bn(h, gamma, beta):
    mu = jnp.mean(h, axis=0)
    var = jnp.var(h, axis=0)
    return (h - mu) / jnp.sqrt(var + 1e-5) * gamma + beta


def kernel(x, edge_index, edge_attr, batch, ptr, g, atom_W, atom_b, edge_W, edge_b, W1, b1, W2, b2, eps, bn_g, bn_b, gate_W1, gate_b1, gate_bn_g, gate_bn_b, gate_W2, gate_b2, h_W1, h_b1, h_W2, h_b2, h_W3, h_b3):
    src = edge_index[1]
    dst = edge_index[0]
    h = x @ atom_W + atom_b
    for l in range(4):
        e = edge_attr @ edge_W[l] + edge_b[l]
        m = jax.nn.relu(h[src] + e)
        aggr = jax.ops.segment_sum(m, dst, num_segments=N)
        z = (1.0 + eps[l]) * h + aggr
        z = _gin_mlp(z, W1[l], b1[l], W2[l], b2[l])
        z = _bn(z, bn_g[l], bn_b[l])
        if l < 3:
            z = jax.nn.relu(z)
        h = z
    gate = h @ gate_W1 + gate_b1
    gate = _bn(gate, gate_bn_g, gate_bn_b)
    gate = jax.nn.relu(gate)
    gate = (gate @ gate_W2 + gate_b2)[:, 0]
    gmax = jax.ops.segment_max(gate, batch, num_segments=B)
    gmax = jnp.where(jnp.isfinite(gmax), gmax, 0.0)
    ex = jnp.exp(gate - gmax[batch])
    denom = jax.ops.segment_sum(ex, batch, num_segments=B)
    alpha = ex / denom[batch]
    h_graph = jax.ops.segment_sum(alpha[:, None] * h, batch, num_segments=B)
    hcat = jnp.concatenate([h_graph, g], axis=1)
    out = jax.nn.relu(hcat @ h_W1 + h_b1)
    out = jax.nn.relu(out @ h_W2 + h_b2)
    out = out @ h_W3 + h_b3
    return out


# speedup vs baseline: 1.5227x; 1.5227x over previous
import functools

import jax
import jax.numpy as jnp
from jax import lax
from jax.experimental import pallas as pl
from jax.experimental.pallas import tpu as pltpu
from jax.experimental.pallas import tpu_sc as plsc

N = 50000
E = 800000
B = 128
EMB = 128
NODE_BLK = 2000
EDGE_BLK = 2048
E_PAD = 800768
ER = E // 128
ER_PAD = E_PAD // 128
NS = 16
SUPER = 8
NSUPER_TOT = ER_PAD // SUPER
RANGE = 12504
NSPR = 12640
WB_ROWS = 784

_mesh = plsc.VectorSubcoreMesh(core_axis_name="c", subcore_axis_name="s")


def _sc_gather(h, src2d):
    NW = 32
    NST = ER // SUPER
    TAIL = ER - NST * SUPER

    @functools.partial(
        pl.kernel,
        out_type=jax.ShapeDtypeStruct((E, EMB), jnp.float32),
        mesh=_mesh,
        scratch_types=[
            pltpu.VMEM((SUPER, 128), jnp.int32),
            pltpu.VMEM((128, EMB), jnp.float32),
            pltpu.SemaphoreType.DMA,
        ],
    )
    def k(h_hbm, i_hbm, o_hbm, idx_v, rows_v, sem):
        cid = lax.axis_index("c")
        sid = lax.axis_index("s")
        wid = sid * 2 + cid

        @pl.loop(0, (NST + NW - 1) // NW)
        def _(g):
            t = g * NW + wid

            @pl.when(t < NST)
            def _():
                pltpu.sync_copy(i_hbm.at[pl.ds(t * SUPER, SUPER)], idx_v)
                for j in range(SUPER):
                    pltpu.async_copy(
                        h_hbm.at[idx_v.at[j]], rows_v, sem).wait()
                    pltpu.sync_copy(
                        rows_v, o_hbm.at[pl.ds((t * SUPER + j) * 128, 128)])

        @pl.when(wid == 0)
        def _():
            pltpu.sync_copy(
                i_hbm.at[pl.ds(NST * SUPER, TAIL)], idx_v.at[pl.ds(0, TAIL)])
            for j in range(TAIL):
                pltpu.async_copy(h_hbm.at[idx_v.at[j]], rows_v, sem).wait()
                pltpu.sync_copy(
                    rows_v, o_hbm.at[pl.ds((NST * SUPER + j) * 128, 128)])

    return k(h, src2d)


def _sc_scatter(m, dst2d):
    NGSUP = (NSUPER_TOT + NS - 1) // NS

    @functools.partial(
        pl.kernel,
        out_type=jax.ShapeDtypeStruct((N, EMB), jnp.float32),
        mesh=_mesh,
        scratch_types=[
            pltpu.VMEM_SHARED((NSPR, EMB), jnp.float32),
            pltpu.VMEM((SUPER, 128), jnp.int32),
            pltpu.VMEM((SUPER, 128), jnp.int32),
            pltpu.VMEM((128, EMB), jnp.float32),
        ],
    )
    def k(m_hbm, d_hbm, o_hbm, acc, idx_v, idxs_v, m_v):
        cid = lax.axis_index("c")
        sid = lax.axis_index("s")

        zvec = jnp.zeros((16,), jnp.float32)
        iota16 = lax.iota(jnp.int32, 16)

        def run_range(r0, nrows):
            @pl.loop(0, 128)
            def _(r):
                @pl.loop(0, EMB // 16)
                def _(c):
                    m_v[r, pl.ds(c * 16, 16)] = zvec

            zbase = sid * (NSPR // NS)

            @pl.loop(0, 6)
            def _(zi):
                pltpu.sync_copy(
                    m_v, acc.at[pl.ds(zbase + zi * 128, 128)])

            pltpu.sync_copy(
                m_v.at[pl.ds(0, (NSPR // NS) - 6 * 128)],
                acc.at[pl.ds(zbase + 6 * 128, (NSPR // NS) - 6 * 128)])
            plsc.subcore_barrier()

            @pl.loop(0, NGSUP)
            def _(g):
                t = g * NS + sid

                @pl.when(t < NSUPER_TOT)
                def _():
                    pltpu.sync_copy(d_hbm.at[pl.ds(t * SUPER, SUPER)], idx_v)
                    for r in range(SUPER):
                        for c in range(8):
                            u = idx_v[r, pl.ds(c * 16, 16)] - r0
                            oob = (u < 0) | (u >= nrows)
                            pad = (RANGE + c * 16) + iota16
                            idxs_v[r, pl.ds(c * 16, 16)] = jnp.where(
                                oob, pad, u)
                    for j in range(SUPER):
                        pltpu.sync_copy(
                            m_hbm.at[pl.ds((t * SUPER + j) * 128, 128)],
                            m_v)
                        pltpu.sync_copy(
                            m_v, acc.at[idxs_v.at[j]], add=True)

            plsc.subcore_barrier()

            last = nrows - (NS - 1) * WB_ROWS

            @pl.when(sid < NS - 1)
            def _():
                pltpu.sync_copy(
                    acc.at[pl.ds(sid * WB_ROWS, WB_ROWS)],
                    o_hbm.at[pl.ds(r0 + sid * WB_ROWS, WB_ROWS)])

            @pl.when(sid == NS - 1)
            def _():
                pltpu.sync_copy(
                    acc.at[pl.ds((NS - 1) * WB_ROWS, last)],
                    o_hbm.at[pl.ds(r0 + (NS - 1) * WB_ROWS, last)])

            plsc.subcore_barrier()

        @pl.when(cid == 0)
        def _():
            run_range(0, RANGE)
            run_range(RANGE, RANGE)

        @pl.when(cid == 1)
        def _():
            run_range(2 * RANGE, RANGE)
            run_range(3 * RANGE, N - 3 * RANGE)

    return k(m, dst2d)


def _edge_body(hs_ref, ea_ref, w_ref, b_ref, o_ref):
    e = jnp.dot(ea_ref[...], w_ref[...], preferred_element_type=jnp.float32)
    o_ref[...] = jnp.maximum(hs_ref[...] + e + b_ref[...], 0.0)


def _edge_kernel(hs, edge_attr, eW, eb):
    return pl.pallas_call(
        _edge_body,
        grid=(E_PAD // EDGE_BLK,),
        in_specs=[
            pl.BlockSpec((EDGE_BLK, EMB), lambda i: (i, 0)),
            pl.BlockSpec((EDGE_BLK, 50), lambda i: (i, 0)),
            pl.BlockSpec((50, EMB), lambda i: (0, 0)),
            pl.BlockSpec((1, EMB), lambda i: (0, 0)),
        ],
        out_specs=pl.BlockSpec((EDGE_BLK, EMB), lambda i: (i, 0)),
        out_shape=jax.ShapeDtypeStruct((E_PAD, EMB), jnp.float32),
    )(hs, edge_attr, eW, eb.reshape(1, EMB))


def _node_a_body(h_ref, a_ref, w1_ref, b1_ref, w2_ref, b2_ref,
                 eps_ref, z2_ref, st_ref, sacc, ssacc):
    i = pl.program_id(0)

    @pl.when(i == 0)
    def _():
        sacc[...] = jnp.zeros_like(sacc)
        ssacc[...] = jnp.zeros_like(ssacc)

    z = (1.0 + eps_ref[0, 0]) * h_ref[...] + a_ref[...]
    z1 = jnp.maximum(
        jnp.dot(z, w1_ref[...], preferred_element_type=jnp.float32)
        + b1_ref[...], 0.0)
    z2 = jnp.dot(z1, w2_ref[...], preferred_element_type=jnp.float32) \
        + b2_ref[...]
    z2_ref[...] = z2
    sacc[...] += jnp.sum(z2, axis=0, keepdims=True)
    ssacc[...] += jnp.sum(z2 * z2, axis=0, keepdims=True)

    @pl.when(i == pl.num_programs(0) - 1)
    def _():
        st_ref[...] = jnp.concatenate([sacc[...], ssacc[...]], axis=0)


def _node_a(h, a, w1, b1, w2, b2, eps_l):
    return pl.pallas_call(
        _node_a_body,
        grid=(N // NODE_BLK,),
        in_specs=[
            pl.BlockSpec((NODE_BLK, EMB), lambda i: (i, 0)),
            pl.BlockSpec((NODE_BLK, EMB), lambda i: (i, 0)),
            pl.BlockSpec((EMB, EMB), lambda i: (0, 0)),
            pl.BlockSpec((1, EMB), lambda i: (0, 0)),
            pl.BlockSpec((EMB, EMB), lambda i: (0, 0)),
            pl.BlockSpec((1, EMB), lambda i: (0, 0)),
            pl.BlockSpec((1, 1), lambda i: (0, 0)),
        ],
        out_specs=[
            pl.BlockSpec((NODE_BLK, EMB), lambda i: (i, 0)),
            pl.BlockSpec((2, EMB), lambda i: (0, 0)),
        ],
        out_shape=[
            jax.ShapeDtypeStruct((N, EMB), jnp.float32),
            jax.ShapeDtypeStruct((2, EMB), jnp.float32),
        ],
        scratch_shapes=[
            pltpu.VMEM((1, EMB), jnp.float32),
            pltpu.VMEM((1, EMB), jnp.float32),
        ],
    )(h, a, w1, b1.reshape(1, EMB), w2, b2.reshape(1, EMB), eps_l)


def _norm_body(z2_ref, st_ref, g_ref, b_ref, o_ref, *, with_relu):
    mu = st_ref[0:1, :] * (1.0 / N)
    var = st_ref[1:2, :] * (1.0 / N) - mu * mu
    inv = lax.rsqrt(var + 1e-5)
    o = (z2_ref[...] - mu) * inv * g_ref[...] + b_ref[...]
    if with_relu:
        o = jnp.maximum(o, 0.0)
    o_ref[...] = o


def _node_b(z2, st, gamma, beta, with_relu):
    return pl.pallas_call(
        functools.partial(_norm_body, with_relu=with_relu),
        grid=(N // NODE_BLK,),
        in_specs=[
            pl.BlockSpec((NODE_BLK, EMB), lambda i: (i, 0)),
            pl.BlockSpec((2, EMB), lambda i: (0, 0)),
            pl.BlockSpec((1, EMB), lambda i: (0, 0)),
            pl.BlockSpec((1, EMB), lambda i: (0, 0)),
        ],
        out_specs=pl.BlockSpec((NODE_BLK, EMB), lambda i: (i, 0)),
        out_shape=jax.ShapeDtypeStruct((N, EMB), jnp.float32),
    )(z2, st, gamma.reshape(1, EMB), beta.reshape(1, EMB))


def _mm_stats_body(h_ref, w_ref, b_ref, t_ref, st_ref, sacc, ssacc):
    i = pl.program_id(0)

    @pl.when(i == 0)
    def _():
        sacc[...] = jnp.zeros_like(sacc)
        ssacc[...] = jnp.zeros_like(ssacc)

    t = jnp.dot(h_ref[...], w_ref[...], preferred_element_type=jnp.float32) \
        + b_ref[...]
    t_ref[...] = t
    sacc[...] += jnp.sum(t, axis=0, keepdims=True)
    ssacc[...] += jnp.sum(t * t, axis=0, keepdims=True)

    @pl.when(i == pl.num_programs(0) - 1)
    def _():
        st_ref[...] = jnp.concatenate([sacc[...], ssacc[...]], axis=0)


def _mm_stats(h, w, b):
    return pl.pallas_call(
        _mm_stats_body,
        grid=(N // NODE_BLK,),
        in_specs=[
            pl.BlockSpec((NODE_BLK, EMB), lambda i: (i, 0)),
            pl.BlockSpec((EMB, EMB), lambda i: (0, 0)),
            pl.BlockSpec((1, EMB), lambda i: (0, 0)),
        ],
        out_specs=[
            pl.BlockSpec((NODE_BLK, EMB), lambda i: (i, 0)),
            pl.BlockSpec((2, EMB), lambda i: (0, 0)),
        ],
        out_shape=[
            jax.ShapeDtypeStruct((N, EMB), jnp.float32),
            jax.ShapeDtypeStruct((2, EMB), jnp.float32),
        ],
        scratch_shapes=[
            pltpu.VMEM((1, EMB), jnp.float32),
            pltpu.VMEM((1, EMB), jnp.float32),
        ],
    )(h, w, b.reshape(1, EMB))


def _gate2_body(t_ref, st_ref, g_ref, be_ref, w2_ref, b2_ref, bat_ref,
                gate_ref, gmax_ref, macc):
    i = pl.program_id(0)

    @pl.when(i == 0)
    def _():
        macc[...] = jnp.full_like(macc, -jnp.inf)

    mu = st_ref[0:1, :] * (1.0 / N)
    var = st_ref[1:2, :] * (1.0 / N) - mu * mu
    inv = lax.rsqrt(var + 1e-5)
    t = jnp.maximum((t_ref[...] - mu) * inv * g_ref[...] + be_ref[...], 0.0)
    gate = lax.dot_general(
        w2_ref[...], t,
        dimension_numbers=(((0,), (1,)), ((), ())),
        preferred_element_type=jnp.float32) + b2_ref[0, 0]
    gate_ref[0, :, :] = gate
    ids = bat_ref[0, :, :]
    onehot = (ids == lax.broadcasted_iota(jnp.int32, (B, NODE_BLK), 0))
    masked = jnp.where(onehot, gate, -jnp.inf)
    macc[...] = jnp.maximum(macc[...], jnp.max(masked, axis=1, keepdims=True))

    @pl.when(i == pl.num_programs(0) - 1)
    def _():
        gmax_ref[...] = jnp.where(
            jnp.isfinite(macc[...]), macc[...], 0.0)


def _gate2(t, st, gamma, beta, w2, b2, batch3d):
    return pl.pallas_call(
        _gate2_body,
        grid=(N // NODE_BLK,),
        in_specs=[
            pl.BlockSpec((NODE_BLK, EMB), lambda i: (i, 0)),
            pl.BlockSpec((2, EMB), lambda i: (0, 0)),
            pl.BlockSpec((1, EMB), lambda i: (0, 0)),
            pl.BlockSpec((1, EMB), lambda i: (0, 0)),
            pl.BlockSpec((EMB, 1), lambda i: (0, 0)),
            pl.BlockSpec((1, 1), lambda i: (0, 0)),
            pl.BlockSpec((1, 1, NODE_BLK), lambda i: (i, 0, 0)),
        ],
        out_specs=[
            pl.BlockSpec((1, 1, NODE_BLK), lambda i: (i, 0, 0)),
            pl.BlockSpec((B, 1), lambda i: (0, 0)),
        ],
        out_shape=[
            jax.ShapeDtypeStruct((N // NODE_BLK, 1, NODE_BLK), jnp.float32),
            jax.ShapeDtypeStruct((B, 1), jnp.float32),
        ],
        scratch_shapes=[pltpu.VMEM((B, 1), jnp.float32)],
    )(t, st, gamma.reshape(1, EMB), beta.reshape(1, EMB), w2,
      b2.reshape(1, 1), batch3d)


def _gate3_body(gate_ref, gmax_ref, bat_ref, ex_ref, den_ref, dacc):
    i = pl.program_id(0)

    @pl.when(i == 0)
    def _():
        dacc[...] = jnp.zeros_like(dacc)

    ids = bat_ref[0, :, :]
    onehot_f = (ids == lax.broadcasted_iota(jnp.int32, (B, NODE_BLK), 0)
                ).astype(jnp.float32)
    gmaxb = lax.dot_general(
        gmax_ref[...], onehot_f,
        dimension_numbers=(((0,), (0,)), ((), ())),
        preferred_element_type=jnp.float32)
    ex = jnp.exp(gate_ref[0, :, :] - gmaxb)
    ex_ref[0, :, :] = ex
    dacc[...] += lax.dot_general(
        onehot_f, ex,
        dimension_numbers=(((1,), (1,)), ((), ())),
        preferred_element_type=jnp.float32)

    @pl.when(i == pl.num_programs(0) - 1)
    def _():
        den_ref[...] = dacc[...]


def _gate3(gate_r, gmax, batch3d):
    return pl.pallas_call(
        _gate3_body,
        grid=(N // NODE_BLK,),
        in_specs=[
            pl.BlockSpec((1, 1, NODE_BLK), lambda i: (i, 0, 0)),
            pl.BlockSpec((B, 1), lambda i: (0, 0)),
            pl.BlockSpec((1, 1, NODE_BLK), lambda i: (i, 0, 0)),
        ],
        out_specs=[
            pl.BlockSpec((1, 1, NODE_BLK), lambda i: (i, 0, 0)),
            pl.BlockSpec((B, 1), lambda i: (0, 0)),
        ],
        out_shape=[
            jax.ShapeDtypeStruct((N // NODE_BLK, 1, NODE_BLK), jnp.float32),
            jax.ShapeDtypeStruct((B, 1), jnp.float32),
        ],
        scratch_shapes=[pltpu.VMEM((B, 1), jnp.float32)],
    )(gate_r, gmax, batch3d)


def _gate4_body(ex_ref, den_ref, h_ref, bat_ref, hg_ref, hacc):
    i = pl.program_id(0)

    @pl.when(i == 0)
    def _():
        hacc[...] = jnp.zeros_like(hacc)

    ids = bat_ref[0, :, :]
    onehot_f = (ids == lax.broadcasted_iota(jnp.int32, (B, NODE_BLK), 0)
                ).astype(jnp.float32)
    denb = lax.dot_general(
        den_ref[...], onehot_f,
        dimension_numbers=(((0,), (0,)), ((), ())),
        preferred_element_type=jnp.float32)
    alpha = ex_ref[0, :, :] / denb
    ow = onehot_f * alpha
    hacc[...] += jnp.dot(ow, h_ref[...],
                         preferred_element_type=jnp.float32)

    @pl.when(i == pl.num_programs(0) - 1)
    def _():
        hg_ref[...] = hacc[...]


def _gate4(ex_r, den, h, batch3d):
    return pl.pallas_call(
        _gate4_body,
        grid=(N // NODE_BLK,),
        in_specs=[
            pl.BlockSpec((1, 1, NODE_BLK), lambda i: (i, 0, 0)),
            pl.BlockSpec((B, 1), lambda i: (0, 0)),
            pl.BlockSpec((NODE_BLK, EMB), lambda i: (i, 0)),
            pl.BlockSpec((1, 1, NODE_BLK), lambda i: (i, 0, 0)),
        ],
        out_specs=pl.BlockSpec((B, EMB), lambda i: (0, 0)),
        out_shape=jax.ShapeDtypeStruct((B, EMB), jnp.float32),
        scratch_shapes=[pltpu.VMEM((B, EMB), jnp.float32)],
    )(ex_r, den, h, batch3d)


def _atom_body(x_ref, w_ref, b_ref, o_ref):
    o_ref[...] = jnp.dot(x_ref[...], w_ref[...],
                         preferred_element_type=jnp.float32) + b_ref[...]


def _atom(x, w, b):
    return pl.pallas_call(
        _atom_body,
        grid=(N // NODE_BLK,),
        in_specs=[
            pl.BlockSpec((NODE_BLK, 92), lambda i: (i, 0)),
            pl.BlockSpec((92, EMB), lambda i: (0, 0)),
            pl.BlockSpec((1, EMB), lambda i: (0, 0)),
        ],
        out_specs=pl.BlockSpec((NODE_BLK, EMB), lambda i: (i, 0)),
        out_shape=jax.ShapeDtypeStruct((N, EMB), jnp.float32),
    )(x, w, b.reshape(1, EMB))


def _head_body(hg_ref, g_ref, w1_ref, b1_ref, w2_ref, b2_ref, w3_ref,
               b3_ref, o_ref):
    hcat = jnp.concatenate([hg_ref[...], g_ref[...]], axis=1)
    o = jnp.maximum(jnp.dot(hcat, w1_ref[...],
                            preferred_element_type=jnp.float32)
                    + b1_ref[...], 0.0)
    o = jnp.maximum(jnp.dot(o, w2_ref[...],
                            preferred_element_type=jnp.float32)
                    + b2_ref[...], 0.0)
    o_ref[...] = jnp.dot(o, w3_ref[...],
                         preferred_element_type=jnp.float32) + b3_ref[...]


def _head(hg, g, w1, b1, w2, b2, w3, b3):
    H0 = EMB + 10
    return pl.pallas_call(
        _head_body,
        in_specs=[
            pl.BlockSpec((B, EMB), lambda: (0, 0)),
            pl.BlockSpec((B, 10), lambda: (0, 0)),
            pl.BlockSpec((H0, 2 * H0), lambda: (0, 0)),
            pl.BlockSpec((1, 2 * H0), lambda: (0, 0)),
            pl.BlockSpec((2 * H0, H0), lambda: (0, 0)),
            pl.BlockSpec((1, H0), lambda: (0, 0)),
            pl.BlockSpec((H0, 1), lambda: (0, 0)),
            pl.BlockSpec((1, 1), lambda: (0, 0)),
        ],
        out_specs=pl.BlockSpec((B, 1), lambda: (0, 0)),
        out_shape=jax.ShapeDtypeStruct((B, 1), jnp.float32),
    )(hg, g, w1, b1.reshape(1, 2 * H0), w2, b2.reshape(1, H0), w3,
      b3.reshape(1, 1))


def kernel(x, edge_index, edge_attr, batch, ptr, g, atom_W, atom_b, edge_W,
           edge_b, W1, b1, W2, b2, eps, bn_g, bn_b, gate_W1, gate_b1,
           gate_bn_g, gate_bn_b, gate_W2, gate_b2, h_W1, h_b1, h_W2, h_b2,
           h_W3, h_b3):
    src2d = edge_index[1].reshape(ER, 128)
    pad_idx = jnp.full((E_PAD - E,), N, jnp.int32)
    dst2d = jnp.concatenate([edge_index[0], pad_idx]).reshape(ER_PAD, 128)
    batch3d = batch.reshape(N // NODE_BLK, 1, NODE_BLK)

    h = _atom(x, atom_W, atom_b)
    for l in range(4):
        hs = _sc_gather(h, src2d)
        m = _edge_kernel(hs, edge_attr, edge_W[l], edge_b[l])
        a = _sc_scatter(m, dst2d)
        z2, st = _node_a(h, a, W1[l], b1[l], W2[l], b2[l],
                         eps[l].reshape(1, 1))
        h = _node_b(z2, st, bn_g[l], bn_b[l], with_relu=(l < 3))

    t, tst = _mm_stats(h, gate_W1, gate_b1)
    gate_r, gmax = _gate2(t, tst, gate_bn_g, gate_bn_b, gate_W2, gate_b2,
                          batch3d)
    ex_r, den = _gate3(gate_r, gmax, batch3d)
    hg = _gate4(ex_r, den, h, batch3d)
    return _head(hg, g, h_W1, h_b1, h_W2, h_b2, h_W3, h_b3)


# speedup vs baseline: 1.5802x; 1.0377x over previous
import functools

import jax
import jax.numpy as jnp
from jax import lax
from jax.experimental import pallas as pl
from jax.experimental.pallas import tpu as pltpu
from jax.experimental.pallas import tpu_sc as plsc

N = 50000
E = 800000
B = 128
EMB = 128
NODE_BLK = 2000
EDGE_BLK = 2048
E_PAD = 800768
ER = E // 128
ER_PAD = E_PAD // 128
NS = 16
SUPER = 8
NSUP = ER_PAD // SUPER
RANGE = 12504
NSPR = 12640
WB_ROWS = 784

_mesh = plsc.VectorSubcoreMesh(core_axis_name="c", subcore_axis_name="s")


def _sc_gather(h, src2d):
    NW = 32
    NST = ER // SUPER
    TAIL = ER - NST * SUPER

    @functools.partial(
        pl.kernel,
        out_type=jax.ShapeDtypeStruct((E, EMB), jnp.float32),
        mesh=_mesh,
        scratch_types=[
            pltpu.VMEM((SUPER, 128), jnp.int32),
            pltpu.VMEM((128, EMB), jnp.float32),
            pltpu.VMEM((128, EMB), jnp.float32),
            pltpu.VMEM((128, EMB), jnp.float32),
            pltpu.VMEM((128, EMB), jnp.float32),
            pltpu.SemaphoreType.DMA,
            pltpu.SemaphoreType.DMA,
            pltpu.SemaphoreType.DMA,
            pltpu.SemaphoreType.DMA,
            pltpu.SemaphoreType.DMA,
            pltpu.SemaphoreType.DMA,
            pltpu.SemaphoreType.DMA,
            pltpu.SemaphoreType.DMA,
        ],
    )
    def k(h_hbm, i_hbm, o_hbm, idx_v, r0, r1, r2, r3,
          g0, g1, g2, g3, w0, w1, w2, w3):
        cid = lax.axis_index("c")
        sid = lax.axis_index("s")
        wid = sid * 2 + cid
        rows = [r0, r1, r2, r3]
        gsem = [g0, g1, g2, g3]
        wsem = [w0, w1, w2, w3]

        @pl.loop(0, (NST + NW - 1) // NW)
        def _(g):
            t = g * NW + wid

            @pl.when(t < NST)
            def _():
                pltpu.sync_copy(i_hbm.at[pl.ds(t * SUPER, SUPER)], idx_v)
                gh = [None] * 4
                wo = [None] * 4
                for j in range(SUPER + 2):
                    if j < SUPER:
                        b = j % 4
                        if wo[b] is not None:
                            wo[b].wait()
                        gh[b] = pltpu.async_copy(
                            h_hbm.at[idx_v.at[j]], rows[b], gsem[b])
                    if j >= 2:
                        p = (j - 2) % 4
                        gh[p].wait()
                        wo[p] = pltpu.async_copy(
                            rows[p],
                            o_hbm.at[pl.ds((t * SUPER + j - 2) * 128, 128)],
                            wsem[p])
                for p in range(4):
                    wo[p].wait()

        @pl.when(wid == 0)
        def _():
            pltpu.sync_copy(
                i_hbm.at[pl.ds(NST * SUPER, TAIL)], idx_v.at[pl.ds(0, TAIL)])
            for j in range(TAIL):
                pltpu.async_copy(h_hbm.at[idx_v.at[j]], r0, g0).wait()
                pltpu.sync_copy(
                    r0, o_hbm.at[pl.ds((NST * SUPER + j) * 128, 128)])

    return k(h, src2d)


def _sc_scatter(m, dst2d):
    @functools.partial(
        pl.kernel,
        out_type=jax.ShapeDtypeStruct((N, EMB), jnp.float32),
        mesh=_mesh,
        scratch_types=[
            pltpu.VMEM_SHARED((NSPR, EMB), jnp.float32),
            pltpu.VMEM((SUPER, 128), jnp.int32),
            pltpu.VMEM((2 * SUPER, 64), jnp.int32),
            pltpu.VMEM((64, EMB), jnp.float32),
            pltpu.VMEM((64, EMB), jnp.float32),
            pltpu.SemaphoreType.DMA,
            pltpu.SemaphoreType.DMA,
        ],
    )
    def k(m_hbm, d_hbm, o_hbm, acc, idx_v, idxs_v, mv0, mv1, sl0, sl1):
        cid = lax.axis_index("c")
        sid = lax.axis_index("s")
        mv = [mv0, mv1]
        sl = [sl0, sl1]

        zvec = jnp.zeros((16,), jnp.float32)
        iota16 = lax.iota(jnp.int32, 16)

        def run_pass(r0, nrows):
            @pl.loop(0, 64)
            def _(r):
                @pl.loop(0, EMB // 16)
                def _(c):
                    mv0[r, pl.ds(c * 16, 16)] = zvec

            zbase = sid * (NSPR // NS)

            @pl.loop(0, 12)
            def _(zi):
                pltpu.sync_copy(mv0, acc.at[pl.ds(zbase + zi * 64, 64)])

            pltpu.sync_copy(
                mv0.at[pl.ds(0, (NSPR // NS) - 12 * 64)],
                acc.at[pl.ds(zbase + 12 * 64, (NSPR // NS) - 12 * 64)])
            plsc.subcore_barrier()

            @pl.loop(0, (NSUP + NS - 1) // NS)
            def _(g):
                t = g * NS + sid

                @pl.when(t < NSUP)
                def _():
                    pltpu.sync_copy(d_hbm.at[pl.ds(t * SUPER, SUPER)], idx_v)
                    for r in range(SUPER):
                        for c in range(8):
                            u = idx_v[r, pl.ds(c * 16, 16)] - r0
                            oob = (u < 0) | (u >= nrows)
                            pad = (RANGE + ((r * 8 + c) % 8) * 16) + iota16
                            idxs_v[2 * r + c // 4, pl.ds((c % 4) * 16, 16)] \
                                = jnp.where(oob, pad, u)
                    lh = [None, None]
                    lh[0] = pltpu.async_copy(
                        m_hbm.at[pl.ds(t * SUPER * 128, 64)], mv0, sl0)
                    for j in range(2 * SUPER):
                        b = j % 2
                        lh[b].wait()
                        if j < 2 * SUPER - 1:
                            lh[1 - b] = pltpu.async_copy(
                                m_hbm.at[pl.ds(t * SUPER * 128
                                               + (j + 1) * 64, 64)],
                                mv[1 - b], sl[1 - b])
                        pltpu.sync_copy(
                            mv[b], acc.at[idxs_v.at[j]], add=True)

            plsc.subcore_barrier()

            last = nrows - (NS - 1) * WB_ROWS

            @pl.when(sid < NS - 1)
            def _():
                pltpu.sync_copy(
                    acc.at[pl.ds(sid * WB_ROWS, WB_ROWS)],
                    o_hbm.at[pl.ds(r0 + sid * WB_ROWS, WB_ROWS)])

            @pl.when(sid == NS - 1)
            def _():
                pltpu.sync_copy(
                    acc.at[pl.ds((NS - 1) * WB_ROWS, last)],
                    o_hbm.at[pl.ds(r0 + (NS - 1) * WB_ROWS, last)])

            plsc.subcore_barrier()

        @pl.when(cid == 0)
        def _():
            run_pass(0, RANGE)
            run_pass(RANGE, RANGE)

        @pl.when(cid == 1)
        def _():
            run_pass(2 * RANGE, RANGE)
            run_pass(3 * RANGE, N - 3 * RANGE)

    return k(m, dst2d)


def _edge_body(hs_ref, ea_ref, w_ref, b_ref, o_ref):
    e = jnp.dot(ea_ref[...], w_ref[...], preferred_element_type=jnp.float32)
    o_ref[...] = jnp.maximum(hs_ref[...] + e + b_ref[...], 0.0)


def _edge_kernel(hs, edge_attr, eW, eb):
    return pl.pallas_call(
        _edge_body,
        grid=(E_PAD // EDGE_BLK,),
        in_specs=[
            pl.BlockSpec((EDGE_BLK, EMB), lambda i: (i, 0)),
            pl.BlockSpec((EDGE_BLK, 50), lambda i: (i, 0)),
            pl.BlockSpec((50, EMB), lambda i: (0, 0)),
            pl.BlockSpec((1, EMB), lambda i: (0, 0)),
        ],
        out_specs=pl.BlockSpec((EDGE_BLK, EMB), lambda i: (i, 0)),
        out_shape=jax.ShapeDtypeStruct((E_PAD, EMB), jnp.float32),
    )(hs, edge_attr, eW, eb.reshape(1, EMB))


def _node_a_body(h_ref, a_ref, w1_ref, b1_ref, w2_ref, b2_ref,
                 eps_ref, z2_ref, st_ref, sacc, ssacc):
    i = pl.program_id(0)

    @pl.when(i == 0)
    def _():
        sacc[...] = jnp.zeros_like(sacc)
        ssacc[...] = jnp.zeros_like(ssacc)

    z = (1.0 + eps_ref[0, 0]) * h_ref[...] + a_ref[...]
    z1 = jnp.maximum(
        jnp.dot(z, w1_ref[...], preferred_element_type=jnp.float32)
        + b1_ref[...], 0.0)
    z2 = jnp.dot(z1, w2_ref[...], preferred_element_type=jnp.float32) \
        + b2_ref[...]
    z2_ref[...] = z2
    sacc[...] += jnp.sum(z2, axis=0, keepdims=True)
    ssacc[...] += jnp.sum(z2 * z2, axis=0, keepdims=True)

    @pl.when(i == pl.num_programs(0) - 1)
    def _():
        st_ref[...] = jnp.concatenate([sacc[...], ssacc[...]], axis=0)


def _node_a(h, a, w1, b1, w2, b2, eps_l):
    return pl.pallas_call(
        _node_a_body,
        grid=(N // NODE_BLK,),
        in_specs=[
            pl.BlockSpec((NODE_BLK, EMB), lambda i: (i, 0)),
            pl.BlockSpec((NODE_BLK, EMB), lambda i: (i, 0)),
            pl.BlockSpec((EMB, EMB), lambda i: (0, 0)),
            pl.BlockSpec((1, EMB), lambda i: (0, 0)),
            pl.BlockSpec((EMB, EMB), lambda i: (0, 0)),
            pl.BlockSpec((1, EMB), lambda i: (0, 0)),
            pl.BlockSpec((1, 1), lambda i: (0, 0)),
        ],
        out_specs=[
            pl.BlockSpec((NODE_BLK, EMB), lambda i: (i, 0)),
            pl.BlockSpec((2, EMB), lambda i: (0, 0)),
        ],
        out_shape=[
            jax.ShapeDtypeStruct((N, EMB), jnp.float32),
            jax.ShapeDtypeStruct((2, EMB), jnp.float32),
        ],
        scratch_shapes=[
            pltpu.VMEM((1, EMB), jnp.float32),
            pltpu.VMEM((1, EMB), jnp.float32),
        ],
    )(h, a, w1, b1.reshape(1, EMB), w2, b2.reshape(1, EMB), eps_l)


def _norm_body(z2_ref, st_ref, g_ref, b_ref, o_ref, *, with_relu):
    mu = st_ref[0:1, :] * (1.0 / N)
    var = st_ref[1:2, :] * (1.0 / N) - mu * mu
    inv = lax.rsqrt(var + 1e-5)
    o = (z2_ref[...] - mu) * inv * g_ref[...] + b_ref[...]
    if with_relu:
        o = jnp.maximum(o, 0.0)
    o_ref[...] = o


def _node_b(z2, st, gamma, beta, with_relu):
    return pl.pallas_call(
        functools.partial(_norm_body, with_relu=with_relu),
        grid=(N // NODE_BLK,),
        in_specs=[
            pl.BlockSpec((NODE_BLK, EMB), lambda i: (i, 0)),
            pl.BlockSpec((2, EMB), lambda i: (0, 0)),
            pl.BlockSpec((1, EMB), lambda i: (0, 0)),
            pl.BlockSpec((1, EMB), lambda i: (0, 0)),
        ],
        out_specs=pl.BlockSpec((NODE_BLK, EMB), lambda i: (i, 0)),
        out_shape=jax.ShapeDtypeStruct((N, EMB), jnp.float32),
    )(z2, st, gamma.reshape(1, EMB), beta.reshape(1, EMB))


def _mm_stats_body(h_ref, w_ref, b_ref, t_ref, st_ref, sacc, ssacc):
    i = pl.program_id(0)

    @pl.when(i == 0)
    def _():
        sacc[...] = jnp.zeros_like(sacc)
        ssacc[...] = jnp.zeros_like(ssacc)

    t = jnp.dot(h_ref[...], w_ref[...], preferred_element_type=jnp.float32) \
        + b_ref[...]
    t_ref[...] = t
    sacc[...] += jnp.sum(t, axis=0, keepdims=True)
    ssacc[...] += jnp.sum(t * t, axis=0, keepdims=True)

    @pl.when(i == pl.num_programs(0) - 1)
    def _():
        st_ref[...] = jnp.concatenate([sacc[...], ssacc[...]], axis=0)


def _mm_stats(h, w, b):
    return pl.pallas_call(
        _mm_stats_body,
        grid=(N // NODE_BLK,),
        in_specs=[
            pl.BlockSpec((NODE_BLK, EMB), lambda i: (i, 0)),
            pl.BlockSpec((EMB, EMB), lambda i: (0, 0)),
            pl.BlockSpec((1, EMB), lambda i: (0, 0)),
        ],
        out_specs=[
            pl.BlockSpec((NODE_BLK, EMB), lambda i: (i, 0)),
            pl.BlockSpec((2, EMB), lambda i: (0, 0)),
        ],
        out_shape=[
            jax.ShapeDtypeStruct((N, EMB), jnp.float32),
            jax.ShapeDtypeStruct((2, EMB), jnp.float32),
        ],
        scratch_shapes=[
            pltpu.VMEM((1, EMB), jnp.float32),
            pltpu.VMEM((1, EMB), jnp.float32),
        ],
    )(h, w, b.reshape(1, EMB))


def _gate2_body(t_ref, st_ref, g_ref, be_ref, w2_ref, b2_ref, bat_ref,
                gate_ref, gmax_ref, macc):
    i = pl.program_id(0)

    @pl.when(i == 0)
    def _():
        macc[...] = jnp.full_like(macc, -jnp.inf)

    mu = st_ref[0:1, :] * (1.0 / N)
    var = st_ref[1:2, :] * (1.0 / N) - mu * mu
    inv = lax.rsqrt(var + 1e-5)
    t = jnp.maximum((t_ref[...] - mu) * inv * g_ref[...] + be_ref[...], 0.0)
    gate = lax.dot_general(
        w2_ref[...], t,
        dimension_numbers=(((0,), (1,)), ((), ())),
        preferred_element_type=jnp.float32) + b2_ref[0, 0]
    gate_ref[0, :, :] = gate
    ids = bat_ref[0, :, :]
    onehot = (ids == lax.broadcasted_iota(jnp.int32, (B, NODE_BLK), 0))
    masked = jnp.where(onehot, gate, -jnp.inf)
    macc[...] = jnp.maximum(macc[...], jnp.max(masked, axis=1, keepdims=True))

    @pl.when(i == pl.num_programs(0) - 1)
    def _():
        gmax_ref[...] = jnp.where(
            jnp.isfinite(macc[...]), macc[...], 0.0)


def _gate2(t, st, gamma, beta, w2, b2, batch3d):
    return pl.pallas_call(
        _gate2_body,
        grid=(N // NODE_BLK,),
        in_specs=[
            pl.BlockSpec((NODE_BLK, EMB), lambda i: (i, 0)),
            pl.BlockSpec((2, EMB), lambda i: (0, 0)),
            pl.BlockSpec((1, EMB), lambda i: (0, 0)),
            pl.BlockSpec((1, EMB), lambda i: (0, 0)),
            pl.BlockSpec((EMB, 1), lambda i: (0, 0)),
            pl.BlockSpec((1, 1), lambda i: (0, 0)),
            pl.BlockSpec((1, 1, NODE_BLK), lambda i: (i, 0, 0)),
        ],
        out_specs=[
            pl.BlockSpec((1, 1, NODE_BLK), lambda i: (i, 0, 0)),
            pl.BlockSpec((B, 1), lambda i: (0, 0)),
        ],
        out_shape=[
            jax.ShapeDtypeStruct((N // NODE_BLK, 1, NODE_BLK), jnp.float32),
            jax.ShapeDtypeStruct((B, 1), jnp.float32),
        ],
        scratch_shapes=[pltpu.VMEM((B, 1), jnp.float32)],
    )(t, st, gamma.reshape(1, EMB), beta.reshape(1, EMB), w2,
      b2.reshape(1, 1), batch3d)


def _gate3_body(gate_ref, gmax_ref, bat_ref, ex_ref, den_ref, dacc):
    i = pl.program_id(0)

    @pl.when(i == 0)
    def _():
        dacc[...] = jnp.zeros_like(dacc)

    ids = bat_ref[0, :, :]
    onehot_f = (ids == lax.broadcasted_iota(jnp.int32, (B, NODE_BLK), 0)
                ).astype(jnp.float32)
    gmaxb = lax.dot_general(
        gmax_ref[...], onehot_f,
        dimension_numbers=(((0,), (0,)), ((), ())),
        preferred_element_type=jnp.float32)
    ex = jnp.exp(gate_ref[0, :, :] - gmaxb)
    ex_ref[0, :, :] = ex
    dacc[...] += lax.dot_general(
        onehot_f, ex,
        dimension_numbers=(((1,), (1,)), ((), ())),
        preferred_element_type=jnp.float32)

    @pl.when(i == pl.num_programs(0) - 1)
    def _():
        den_ref[...] = dacc[...]


def _gate3(gate_r, gmax, batch3d):
    return pl.pallas_call(
        _gate3_body,
        grid=(N // NODE_BLK,),
        in_specs=[
            pl.BlockSpec((1, 1, NODE_BLK), lambda i: (i, 0, 0)),
            pl.BlockSpec((B, 1), lambda i: (0, 0)),
            pl.BlockSpec((1, 1, NODE_BLK), lambda i: (i, 0, 0)),
        ],
        out_specs=[
            pl.BlockSpec((1, 1, NODE_BLK), lambda i: (i, 0, 0)),
            pl.BlockSpec((B, 1), lambda i: (0, 0)),
        ],
        out_shape=[
            jax.ShapeDtypeStruct((N // NODE_BLK, 1, NODE_BLK), jnp.float32),
            jax.ShapeDtypeStruct((B, 1), jnp.float32),
        ],
        scratch_shapes=[pltpu.VMEM((B, 1), jnp.float32)],
    )(gate_r, gmax, batch3d)


def _gate4_body(ex_ref, den_ref, h_ref, bat_ref, hg_ref, hacc):
    i = pl.program_id(0)

    @pl.when(i == 0)
    def _():
        hacc[...] = jnp.zeros_like(hacc)

    ids = bat_ref[0, :, :]
    onehot_f = (ids == lax.broadcasted_iota(jnp.int32, (B, NODE_BLK), 0)
                ).astype(jnp.float32)
    denb = lax.dot_general(
        den_ref[...], onehot_f,
        dimension_numbers=(((0,), (0,)), ((), ())),
        preferred_element_type=jnp.float32)
    alpha = ex_ref[0, :, :] / denb
    ow = onehot_f * alpha
    hacc[...] += jnp.dot(ow, h_ref[...],
                         preferred_element_type=jnp.float32)

    @pl.when(i == pl.num_programs(0) - 1)
    def _():
        hg_ref[...] = hacc[...]


def _gate4(ex_r, den, h, batch3d):
    return pl.pallas_call(
        _gate4_body,
        grid=(N // NODE_BLK,),
        in_specs=[
            pl.BlockSpec((1, 1, NODE_BLK), lambda i: (i, 0, 0)),
            pl.BlockSpec((B, 1), lambda i: (0, 0)),
            pl.BlockSpec((NODE_BLK, EMB), lambda i: (i, 0)),
            pl.BlockSpec((1, 1, NODE_BLK), lambda i: (i, 0, 0)),
        ],
        out_specs=pl.BlockSpec((B, EMB), lambda i: (0, 0)),
        out_shape=jax.ShapeDtypeStruct((B, EMB), jnp.float32),
        scratch_shapes=[pltpu.VMEM((B, EMB), jnp.float32)],
    )(ex_r, den, h, batch3d)


def _atom_body(x_ref, w_ref, b_ref, o_ref):
    o_ref[...] = jnp.dot(x_ref[...], w_ref[...],
                         preferred_element_type=jnp.float32) + b_ref[...]


def _atom(x, w, b):
    return pl.pallas_call(
        _atom_body,
        grid=(N // NODE_BLK,),
        in_specs=[
            pl.BlockSpec((NODE_BLK, 92), lambda i: (i, 0)),
            pl.BlockSpec((92, EMB), lambda i: (0, 0)),
            pl.BlockSpec((1, EMB), lambda i: (0, 0)),
        ],
        out_specs=pl.BlockSpec((NODE_BLK, EMB), lambda i: (i, 0)),
        out_shape=jax.ShapeDtypeStruct((N, EMB), jnp.float32),
    )(x, w, b.reshape(1, EMB))


def _head_body(hg_ref, g_ref, w1_ref, b1_ref, w2_ref, b2_ref, w3_ref,
               b3_ref, o_ref):
    hcat = jnp.concatenate([hg_ref[...], g_ref[...]], axis=1)
    o = jnp.maximum(jnp.dot(hcat, w1_ref[...],
                            preferred_element_type=jnp.float32)
                    + b1_ref[...], 0.0)
    o = jnp.maximum(jnp.dot(o, w2_ref[...],
                            preferred_element_type=jnp.float32)
                    + b2_ref[...], 0.0)
    o_ref[...] = jnp.dot(o, w3_ref[...],
                         preferred_element_type=jnp.float32) + b3_ref[...]


def _head(hg, g, w1, b1, w2, b2, w3, b3):
    H0 = EMB + 10
    return pl.pallas_call(
        _head_body,
        in_specs=[
            pl.BlockSpec((B, EMB), lambda: (0, 0)),
            pl.BlockSpec((B, 10), lambda: (0, 0)),
            pl.BlockSpec((H0, 2 * H0), lambda: (0, 0)),
            pl.BlockSpec((1, 2 * H0), lambda: (0, 0)),
            pl.BlockSpec((2 * H0, H0), lambda: (0, 0)),
            pl.BlockSpec((1, H0), lambda: (0, 0)),
            pl.BlockSpec((H0, 1), lambda: (0, 0)),
            pl.BlockSpec((1, 1), lambda: (0, 0)),
        ],
        out_specs=pl.BlockSpec((B, 1), lambda: (0, 0)),
        out_shape=jax.ShapeDtypeStruct((B, 1), jnp.float32),
    )(hg, g, w1, b1.reshape(1, 2 * H0), w2, b2.reshape(1, H0), w3,
      b3.reshape(1, 1))


def kernel(x, edge_index, edge_attr, batch, ptr, g, atom_W, atom_b, edge_W,
           edge_b, W1, b1, W2, b2, eps, bn_g, bn_b, gate_W1, gate_b1,
           gate_bn_g, gate_bn_b, gate_W2, gate_b2, h_W1, h_b1, h_W2, h_b2,
           h_W3, h_b3):
    src2d = edge_index[1].reshape(ER, 128)
    pad_idx = jnp.full((E_PAD - E,), N, jnp.int32)
    dst2d = jnp.concatenate([edge_index[0], pad_idx]).reshape(ER_PAD, 128)
    batch3d = batch.reshape(N // NODE_BLK, 1, NODE_BLK)

    h = _atom(x, atom_W, atom_b)
    for l in range(4):
        hs = _sc_gather(h, src2d)
        m = _edge_kernel(hs, edge_attr, edge_W[l], edge_b[l])
        a = _sc_scatter(m, dst2d)
        z2, st = _node_a(h, a, W1[l], b1[l], W2[l], b2[l],
                         eps[l].reshape(1, 1))
        h = _node_b(z2, st, bn_g[l], bn_b[l], with_relu=(l < 3))

    t, tst = _mm_stats(h, gate_W1, gate_b1)
    gate_r, gmax = _gate2(t, tst, gate_bn_g, gate_bn_b, gate_W2, gate_b2,
                          batch3d)
    ex_r, den = _gate3(gate_r, gmax, batch3d)
    hg = _gate4(ex_r, den, h, batch3d)
    return _head(hg, g, h_W1, h_b1, h_W2, h_b2, h_W3, h_b3)


# speedup vs baseline: 1.6443x; 1.0405x over previous
import functools

import jax
import jax.numpy as jnp
from jax import lax
from jax.experimental import pallas as pl
from jax.experimental.pallas import tpu as pltpu
from jax.experimental.pallas import tpu_sc as plsc

N = 50000
E = 800000
B = 128
EMB = 128
NODE_BLK = 2000
E_HALF = E // 2
EH_PAD = 400384
ERH = E_HALF // 128
ERH_PAD = EH_PAD // 128
EDGE_BLK = 2000
NS = 16
SUPER = 8
NSUPH = ERH_PAD // SUPER
RANGE = 12504
NSPR = 12640
WB_ROWS = 784

_mesh = plsc.VectorSubcoreMesh(core_axis_name="c", subcore_axis_name="s")


def _sc_gather(h, src2d):
    NW = 32
    NST = ERH // SUPER
    TAIL = ERH - NST * SUPER

    @functools.partial(
        pl.kernel,
        out_type=jax.ShapeDtypeStruct((E_HALF, EMB), jnp.float32),
        mesh=_mesh,
        scratch_types=[
            pltpu.VMEM((SUPER, 128), jnp.int32),
            pltpu.VMEM((128, EMB), jnp.float32),
            pltpu.VMEM((128, EMB), jnp.float32),
            pltpu.VMEM((128, EMB), jnp.float32),
            pltpu.VMEM((128, EMB), jnp.float32),
            pltpu.SemaphoreType.DMA,
            pltpu.SemaphoreType.DMA,
            pltpu.SemaphoreType.DMA,
            pltpu.SemaphoreType.DMA,
            pltpu.SemaphoreType.DMA,
            pltpu.SemaphoreType.DMA,
            pltpu.SemaphoreType.DMA,
            pltpu.SemaphoreType.DMA,
        ],
    )
    def k(h_hbm, i_hbm, o_hbm, idx_v, r0, r1, r2, r3,
          g0, g1, g2, g3, w0, w1, w2, w3):
        cid = lax.axis_index("c")
        sid = lax.axis_index("s")
        wid = sid * 2 + cid
        rows = [r0, r1, r2, r3]
        gsem = [g0, g1, g2, g3]
        wsem = [w0, w1, w2, w3]

        @pl.loop(0, (NST + NW - 1) // NW)
        def _(g):
            t = g * NW + wid

            @pl.when(t < NST)
            def _():
                pltpu.sync_copy(i_hbm.at[pl.ds(t * SUPER, SUPER)], idx_v)
                gh = [None] * 4
                wo = [None] * 4
                for j in range(SUPER + 2):
                    if j < SUPER:
                        b = j % 4
                        if wo[b] is not None:
                            wo[b].wait()
                        gh[b] = pltpu.async_copy(
                            h_hbm.at[idx_v.at[j]], rows[b], gsem[b])
                    if j >= 2:
                        p = (j - 2) % 4
                        gh[p].wait()
                        wo[p] = pltpu.async_copy(
                            rows[p],
                            o_hbm.at[pl.ds((t * SUPER + j - 2) * 128, 128)],
                            wsem[p])
                for p in range(4):
                    wo[p].wait()

        @pl.when(wid == 0)
        def _():
            pltpu.sync_copy(
                i_hbm.at[pl.ds(NST * SUPER, TAIL)], idx_v.at[pl.ds(0, TAIL)])
            for j in range(TAIL):
                pltpu.async_copy(h_hbm.at[idx_v.at[j]], r0, g0).wait()
                pltpu.sync_copy(
                    r0, o_hbm.at[pl.ds((NST * SUPER + j) * 128, 128)])

    return k(h, src2d)


def _sc_scatter(m, dst2d):
    @functools.partial(
        pl.kernel,
        out_type=jax.ShapeDtypeStruct((N, EMB), jnp.float32),
        mesh=_mesh,
        scratch_types=[
            pltpu.VMEM_SHARED((NSPR, EMB), jnp.float32),
            pltpu.VMEM((SUPER, 128), jnp.int32),
            pltpu.VMEM((2 * SUPER, 64), jnp.int32),
            pltpu.VMEM((64, EMB), jnp.float32),
            pltpu.VMEM((64, EMB), jnp.float32),
            pltpu.SemaphoreType.DMA,
            pltpu.SemaphoreType.DMA,
        ],
    )
    def k(m_hbm, d_hbm, o_hbm, acc, idx_v, idxs_v, mv0, mv1, sl0, sl1):
        cid = lax.axis_index("c")
        sid = lax.axis_index("s")
        mv = [mv0, mv1]
        sl = [sl0, sl1]

        zvec = jnp.zeros((16,), jnp.float32)
        iota16 = lax.iota(jnp.int32, 16)

        def run_pass(r0, nrows):
            @pl.loop(0, 64)
            def _(r):
                @pl.loop(0, EMB // 16)
                def _(c):
                    mv0[r, pl.ds(c * 16, 16)] = zvec

            zbase = sid * (NSPR // NS)

            @pl.loop(0, 12)
            def _(zi):
                pltpu.sync_copy(mv0, acc.at[pl.ds(zbase + zi * 64, 64)])

            pltpu.sync_copy(
                mv0.at[pl.ds(0, (NSPR // NS) - 12 * 64)],
                acc.at[pl.ds(zbase + 12 * 64, (NSPR // NS) - 12 * 64)])
            plsc.subcore_barrier()

            @pl.loop(0, (NSUPH + NS - 1) // NS)
            def _(g):
                t = g * NS + sid

                @pl.when(t < NSUPH)
                def _():
                    pltpu.sync_copy(d_hbm.at[pl.ds(t * SUPER, SUPER)], idx_v)
                    for r in range(SUPER):
                        for c in range(8):
                            u = idx_v[r, pl.ds(c * 16, 16)] - r0
                            oob = (u < 0) | (u >= nrows)
                            pad = (RANGE + ((r * 8 + c) % 8) * 16) + iota16
                            idxs_v[2 * r + c // 4, pl.ds((c % 4) * 16, 16)] \
                                = jnp.where(oob, pad, u)
                    lh = [None, None]
                    lh[0] = pltpu.async_copy(
                        m_hbm.at[pl.ds(t * SUPER * 128, 64)], mv0, sl0)
                    for j in range(2 * SUPER):
                        b = j % 2
                        lh[b].wait()
                        if j < 2 * SUPER - 1:
                            lh[1 - b] = pltpu.async_copy(
                                m_hbm.at[pl.ds(t * SUPER * 128
                                               + (j + 1) * 64, 64)],
                                mv[1 - b], sl[1 - b])
                        pltpu.sync_copy(
                            mv[b], acc.at[idxs_v.at[j]], add=True)

            plsc.subcore_barrier()

            last = nrows - (NS - 1) * WB_ROWS

            @pl.when(sid < NS - 1)
            def _():
                pltpu.sync_copy(
                    acc.at[pl.ds(sid * WB_ROWS, WB_ROWS)],
                    o_hbm.at[pl.ds(r0 + sid * WB_ROWS, WB_ROWS)])

            @pl.when(sid == NS - 1)
            def _():
                pltpu.sync_copy(
                    acc.at[pl.ds((NS - 1) * WB_ROWS, last)],
                    o_hbm.at[pl.ds(r0 + (NS - 1) * WB_ROWS, last)])

            plsc.subcore_barrier()

        @pl.when(cid == 0)
        def _():
            run_pass(0, RANGE)
            run_pass(RANGE, RANGE)

        @pl.when(cid == 1)
        def _():
            run_pass(2 * RANGE, RANGE)
            run_pass(3 * RANGE, N - 3 * RANGE)

    return k(m, dst2d)


def _edge_body(hs_ref, ea_ref, w_ref, b_ref, o_ref):
    e = jnp.dot(ea_ref[...], w_ref[...], preferred_element_type=jnp.float32)
    o_ref[...] = jnp.maximum(hs_ref[...] + e + b_ref[...], 0.0)


def _edge_kernel(hs, edge_attr, eW, eb, half):
    off = half * (E_HALF // EDGE_BLK)
    return pl.pallas_call(
        _edge_body,
        grid=(E_HALF // EDGE_BLK,),
        in_specs=[
            pl.BlockSpec((EDGE_BLK, EMB), lambda i: (i, 0)),
            pl.BlockSpec((EDGE_BLK, 50), lambda i: (i + off, 0)),
            pl.BlockSpec((50, EMB), lambda i: (0, 0)),
            pl.BlockSpec((1, EMB), lambda i: (0, 0)),
        ],
        out_specs=pl.BlockSpec((EDGE_BLK, EMB), lambda i: (i, 0)),
        out_shape=jax.ShapeDtypeStruct((EH_PAD, EMB), jnp.float32),
    )(hs, edge_attr, eW, eb.reshape(1, EMB))


def _node_a_body(h_ref, aA_ref, aB_ref, w1_ref, b1_ref, w2_ref, b2_ref,
                 eps_ref, z2_ref, st_ref, sacc, ssacc):
    i = pl.program_id(0)

    @pl.when(i == 0)
    def _():
        sacc[...] = jnp.zeros_like(sacc)
        ssacc[...] = jnp.zeros_like(ssacc)

    z = (1.0 + eps_ref[0, 0]) * h_ref[...] + aA_ref[...] + aB_ref[...]
    z1 = jnp.maximum(
        jnp.dot(z, w1_ref[...], preferred_element_type=jnp.float32)
        + b1_ref[...], 0.0)
    z2 = jnp.dot(z1, w2_ref[...], preferred_element_type=jnp.float32) \
        + b2_ref[...]
    z2_ref[...] = z2
    sacc[...] += jnp.sum(z2, axis=0, keepdims=True)
    ssacc[...] += jnp.sum(z2 * z2, axis=0, keepdims=True)

    @pl.when(i == pl.num_programs(0) - 1)
    def _():
        st_ref[...] = jnp.concatenate([sacc[...], ssacc[...]], axis=0)


def _node_a(h, aA, aB, w1, b1, w2, b2, eps_l):
    return pl.pallas_call(
        _node_a_body,
        grid=(N // NODE_BLK,),
        in_specs=[
            pl.BlockSpec((NODE_BLK, EMB), lambda i: (i, 0)),
            pl.BlockSpec((NODE_BLK, EMB), lambda i: (i, 0)),
            pl.BlockSpec((NODE_BLK, EMB), lambda i: (i, 0)),
            pl.BlockSpec((EMB, EMB), lambda i: (0, 0)),
            pl.BlockSpec((1, EMB), lambda i: (0, 0)),
            pl.BlockSpec((EMB, EMB), lambda i: (0, 0)),
            pl.BlockSpec((1, EMB), lambda i: (0, 0)),
            pl.BlockSpec((1, 1), lambda i: (0, 0)),
        ],
        out_specs=[
            pl.BlockSpec((NODE_BLK, EMB), lambda i: (i, 0)),
            pl.BlockSpec((2, EMB), lambda i: (0, 0)),
        ],
        out_shape=[
            jax.ShapeDtypeStruct((N, EMB), jnp.float32),
            jax.ShapeDtypeStruct((2, EMB), jnp.float32),
        ],
        scratch_shapes=[
            pltpu.VMEM((1, EMB), jnp.float32),
            pltpu.VMEM((1, EMB), jnp.float32),
        ],
    )(h, aA, aB, w1, b1.reshape(1, EMB), w2, b2.reshape(1, EMB), eps_l)


def _norm_body(z2_ref, st_ref, g_ref, b_ref, o_ref, *, with_relu):
    mu = st_ref[0:1, :] * (1.0 / N)
    var = st_ref[1:2, :] * (1.0 / N) - mu * mu
    inv = lax.rsqrt(var + 1e-5)
    o = (z2_ref[...] - mu) * inv * g_ref[...] + b_ref[...]
    if with_relu:
        o = jnp.maximum(o, 0.0)
    o_ref[...] = o


def _node_b(z2, st, gamma, beta, with_relu):
    return pl.pallas_call(
        functools.partial(_norm_body, with_relu=with_relu),
        grid=(N // NODE_BLK,),
        in_specs=[
            pl.BlockSpec((NODE_BLK, EMB), lambda i: (i, 0)),
            pl.BlockSpec((2, EMB), lambda i: (0, 0)),
            pl.BlockSpec((1, EMB), lambda i: (0, 0)),
            pl.BlockSpec((1, EMB), lambda i: (0, 0)),
        ],
        out_specs=pl.BlockSpec((NODE_BLK, EMB), lambda i: (i, 0)),
        out_shape=jax.ShapeDtypeStruct((N, EMB), jnp.float32),
    )(z2, st, gamma.reshape(1, EMB), beta.reshape(1, EMB))


def _mm_stats_body(h_ref, w_ref, b_ref, t_ref, st_ref, sacc, ssacc):
    i = pl.program_id(0)

    @pl.when(i == 0)
    def _():
        sacc[...] = jnp.zeros_like(sacc)
        ssacc[...] = jnp.zeros_like(ssacc)

    t = jnp.dot(h_ref[...], w_ref[...], preferred_element_type=jnp.float32) \
        + b_ref[...]
    t_ref[...] = t
    sacc[...] += jnp.sum(t, axis=0, keepdims=True)
    ssacc[...] += jnp.sum(t * t, axis=0, keepdims=True)

    @pl.when(i == pl.num_programs(0) - 1)
    def _():
        st_ref[...] = jnp.concatenate([sacc[...], ssacc[...]], axis=0)


def _mm_stats(h, w, b):
    return pl.pallas_call(
        _mm_stats_body,
        grid=(N // NODE_BLK,),
        in_specs=[
            pl.BlockSpec((NODE_BLK, EMB), lambda i: (i, 0)),
            pl.BlockSpec((EMB, EMB), lambda i: (0, 0)),
            pl.BlockSpec((1, EMB), lambda i: (0, 0)),
        ],
        out_specs=[
            pl.BlockSpec((NODE_BLK, EMB), lambda i: (i, 0)),
            pl.BlockSpec((2, EMB), lambda i: (0, 0)),
        ],
        out_shape=[
            jax.ShapeDtypeStruct((N, EMB), jnp.float32),
            jax.ShapeDtypeStruct((2, EMB), jnp.float32),
        ],
        scratch_shapes=[
            pltpu.VMEM((1, EMB), jnp.float32),
            pltpu.VMEM((1, EMB), jnp.float32),
        ],
    )(h, w, b.reshape(1, EMB))


def _gate2_body(t_ref, st_ref, g_ref, be_ref, w2_ref, b2_ref, bat_ref,
                gate_ref, gmax_ref, macc):
    i = pl.program_id(0)

    @pl.when(i == 0)
    def _():
        macc[...] = jnp.full_like(macc, -jnp.inf)

    mu = st_ref[0:1, :] * (1.0 / N)
    var = st_ref[1:2, :] * (1.0 / N) - mu * mu
    inv = lax.rsqrt(var + 1e-5)
    t = jnp.maximum((t_ref[...] - mu) * inv * g_ref[...] + be_ref[...], 0.0)
    gate = lax.dot_general(
        w2_ref[...], t,
        dimension_numbers=(((0,), (1,)), ((), ())),
        preferred_element_type=jnp.float32) + b2_ref[0, 0]
    gate_ref[0, :, :] = gate
    ids = bat_ref[0, :, :]
    onehot = (ids == lax.broadcasted_iota(jnp.int32, (B, NODE_BLK), 0))
    masked = jnp.where(onehot, gate, -jnp.inf)
    macc[...] = jnp.maximum(macc[...], jnp.max(masked, axis=1, keepdims=True))

    @pl.when(i == pl.num_programs(0) - 1)
    def _():
        gmax_ref[...] = jnp.where(
            jnp.isfinite(macc[...]), macc[...], 0.0)


def _gate2(t, st, gamma, beta, w2, b2, batch3d):
    return pl.pallas_call(
        _gate2_body,
        grid=(N // NODE_BLK,),
        in_specs=[
            pl.BlockSpec((NODE_BLK, EMB), lambda i: (i, 0)),
            pl.BlockSpec((2, EMB), lambda i: (0, 0)),
            pl.BlockSpec((1, EMB), lambda i: (0, 0)),
            pl.BlockSpec((1, EMB), lambda i: (0, 0)),
            pl.BlockSpec((EMB, 1), lambda i: (0, 0)),
            pl.BlockSpec((1, 1), lambda i: (0, 0)),
            pl.BlockSpec((1, 1, NODE_BLK), lambda i: (i, 0, 0)),
        ],
        out_specs=[
            pl.BlockSpec((1, 1, NODE_BLK), lambda i: (i, 0, 0)),
            pl.BlockSpec((B, 1), lambda i: (0, 0)),
        ],
        out_shape=[
            jax.ShapeDtypeStruct((N // NODE_BLK, 1, NODE_BLK), jnp.float32),
            jax.ShapeDtypeStruct((B, 1), jnp.float32),
        ],
        scratch_shapes=[pltpu.VMEM((B, 1), jnp.float32)],
    )(t, st, gamma.reshape(1, EMB), beta.reshape(1, EMB), w2,
      b2.reshape(1, 1), batch3d)


def _gate3_body(gate_ref, gmax_ref, bat_ref, ex_ref, den_ref, dacc):
    i = pl.program_id(0)

    @pl.when(i == 0)
    def _():
        dacc[...] = jnp.zeros_like(dacc)

    ids = bat_ref[0, :, :]
    onehot_f = (ids == lax.broadcasted_iota(jnp.int32, (B, NODE_BLK), 0)
                ).astype(jnp.float32)
    gmaxb = lax.dot_general(
        gmax_ref[...], onehot_f,
        dimension_numbers=(((0,), (0,)), ((), ())),
        preferred_element_type=jnp.float32)
    ex = jnp.exp(gate_ref[0, :, :] - gmaxb)
    ex_ref[0, :, :] = ex
    dacc[...] += lax.dot_general(
        onehot_f, ex,
        dimension_numbers=(((1,), (1,)), ((), ())),
        preferred_element_type=jnp.float32)

    @pl.when(i == pl.num_programs(0) - 1)
    def _():
        den_ref[...] = dacc[...]


def _gate3(gate_r, gmax, batch3d):
    return pl.pallas_call(
        _gate3_body,
        grid=(N // NODE_BLK,),
        in_specs=[
            pl.BlockSpec((1, 1, NODE_BLK), lambda i: (i, 0, 0)),
            pl.BlockSpec((B, 1), lambda i: (0, 0)),
            pl.BlockSpec((1, 1, NODE_BLK), lambda i: (i, 0, 0)),
        ],
        out_specs=[
            pl.BlockSpec((1, 1, NODE_BLK), lambda i: (i, 0, 0)),
            pl.BlockSpec((B, 1), lambda i: (0, 0)),
        ],
        out_shape=[
            jax.ShapeDtypeStruct((N // NODE_BLK, 1, NODE_BLK), jnp.float32),
            jax.ShapeDtypeStruct((B, 1), jnp.float32),
        ],
        scratch_shapes=[pltpu.VMEM((B, 1), jnp.float32)],
    )(gate_r, gmax, batch3d)


def _gate4_body(ex_ref, den_ref, h_ref, bat_ref, hg_ref, hacc):
    i = pl.program_id(0)

    @pl.when(i == 0)
    def _():
        hacc[...] = jnp.zeros_like(hacc)

    ids = bat_ref[0, :, :]
    onehot_f = (ids == lax.broadcasted_iota(jnp.int32, (B, NODE_BLK), 0)
                ).astype(jnp.float32)
    denb = lax.dot_general(
        den_ref[...], onehot_f,
        dimension_numbers=(((0,), (0,)), ((), ())),
        preferred_element_type=jnp.float32)
    alpha = ex_ref[0, :, :] / denb
    ow = onehot_f * alpha
    hacc[...] += jnp.dot(ow, h_ref[...],
                         preferred_element_type=jnp.float32)

    @pl.when(i == pl.num_programs(0) - 1)
    def _():
        hg_ref[...] = hacc[...]


def _gate4(ex_r, den, h, batch3d):
    return pl.pallas_call(
        _gate4_body,
        grid=(N // NODE_BLK,),
        in_specs=[
            pl.BlockSpec((1, 1, NODE_BLK), lambda i: (i, 0, 0)),
            pl.BlockSpec((B, 1), lambda i: (0, 0)),
            pl.BlockSpec((NODE_BLK, EMB), lambda i: (i, 0)),
            pl.BlockSpec((1, 1, NODE_BLK), lambda i: (i, 0, 0)),
        ],
        out_specs=pl.BlockSpec((B, EMB), lambda i: (0, 0)),
        out_shape=jax.ShapeDtypeStruct((B, EMB), jnp.float32),
        scratch_shapes=[pltpu.VMEM((B, EMB), jnp.float32)],
    )(ex_r, den, h, batch3d)


def _atom_body(x_ref, w_ref, b_ref, o_ref):
    o_ref[...] = jnp.dot(x_ref[...], w_ref[...],
                         preferred_element_type=jnp.float32) + b_ref[...]


def _atom(x, w, b):
    return pl.pallas_call(
        _atom_body,
        grid=(N // NODE_BLK,),
        in_specs=[
            pl.BlockSpec((NODE_BLK, 92), lambda i: (i, 0)),
            pl.BlockSpec((92, EMB), lambda i: (0, 0)),
            pl.BlockSpec((1, EMB), lambda i: (0, 0)),
        ],
        out_specs=pl.BlockSpec((NODE_BLK, EMB), lambda i: (i, 0)),
        out_shape=jax.ShapeDtypeStruct((N, EMB), jnp.float32),
    )(x, w, b.reshape(1, EMB))


def _head_body(hg_ref, g_ref, w1_ref, b1_ref, w2_ref, b2_ref, w3_ref,
               b3_ref, o_ref):
    hcat = jnp.concatenate([hg_ref[...], g_ref[...]], axis=1)
    o = jnp.maximum(jnp.dot(hcat, w1_ref[...],
                            preferred_element_type=jnp.float32)
                    + b1_ref[...], 0.0)
    o = jnp.maximum(jnp.dot(o, w2_ref[...],
                            preferred_element_type=jnp.float32)
                    + b2_ref[...], 0.0)
    o_ref[...] = jnp.dot(o, w3_ref[...],
                         preferred_element_type=jnp.float32) + b3_ref[...]


def _head(hg, g, w1, b1, w2, b2, w3, b3):
    H0 = EMB + 10
    return pl.pallas_call(
        _head_body,
        in_specs=[
            pl.BlockSpec((B, EMB), lambda: (0, 0)),
            pl.BlockSpec((B, 10), lambda: (0, 0)),
            pl.BlockSpec((H0, 2 * H0), lambda: (0, 0)),
            pl.BlockSpec((1, 2 * H0), lambda: (0, 0)),
            pl.BlockSpec((2 * H0, H0), lambda: (0, 0)),
            pl.BlockSpec((1, H0), lambda: (0, 0)),
            pl.BlockSpec((H0, 1), lambda: (0, 0)),
            pl.BlockSpec((1, 1), lambda: (0, 0)),
        ],
        out_specs=pl.BlockSpec((B, 1), lambda: (0, 0)),
        out_shape=jax.ShapeDtypeStruct((B, 1), jnp.float32),
    )(hg, g, w1, b1.reshape(1, 2 * H0), w2, b2.reshape(1, H0), w3,
      b3.reshape(1, 1))


def kernel(x, edge_index, edge_attr, batch, ptr, g, atom_W, atom_b, edge_W,
           edge_b, W1, b1, W2, b2, eps, bn_g, bn_b, gate_W1, gate_b1,
           gate_bn_g, gate_bn_b, gate_W2, gate_b2, h_W1, h_b1, h_W2, h_b2,
           h_W3, h_b3):
    src = edge_index[1]
    dst = edge_index[0]
    srcA = src[:E_HALF].reshape(ERH, 128)
    srcB = src[E_HALF:].reshape(ERH, 128)
    pad_idx = jnp.full((EH_PAD - E_HALF,), N, jnp.int32)
    dstA = jnp.concatenate([dst[:E_HALF], pad_idx]).reshape(ERH_PAD, 128)
    dstB = jnp.concatenate([dst[E_HALF:], pad_idx]).reshape(ERH_PAD, 128)
    batch3d = batch.reshape(N // NODE_BLK, 1, NODE_BLK)

    h = _atom(x, atom_W, atom_b)
    for l in range(4):
        hsA = _sc_gather(h, srcA)
        mA = _edge_kernel(hsA, edge_attr, edge_W[l], edge_b[l], 0)
        hsB = _sc_gather(h, srcB)
        mB = _edge_kernel(hsB, edge_attr, edge_W[l], edge_b[l], 1)
        aA = _sc_scatter(mA, dstA)
        aB = _sc_scatter(mB, dstB)
        z2, st = _node_a(h, aA, aB, W1[l], b1[l], W2[l], b2[l],
                         eps[l].reshape(1, 1))
        h = _node_b(z2, st, bn_g[l], bn_b[l], with_relu=(l < 3))

    t, tst = _mm_stats(h, gate_W1, gate_b1)
    gate_r, gmax = _gate2(t, tst, gate_bn_g, gate_bn_b, gate_W2, gate_b2,
                          batch3d)
    ex_r, den = _gate3(gate_r, gmax, batch3d)
    hg = _gate4(ex_r, den, h, batch3d)
    return _head(hg, g, h_W1, h_b1, h_W2, h_b2, h_W3, h_b3)


# speedup vs baseline: 2.1910x; 1.3325x over previous
import functools

import jax
import jax.numpy as jnp
from jax import lax
from jax.experimental import pallas as pl
from jax.experimental.pallas import tpu as pltpu
from jax.experimental.pallas import tpu_sc as plsc

N = 50000
E = 800000
B = 128
EMB = 128
NODE_BLK = 2000
E_HALF = E // 2
EH_PAD = 400384
ERH = E_HALF // 128
ERH_PAD = EH_PAD // 128
EDGE_BLK = 2000
NS = 16
SUPER = 8
NSUPH = ERH_PAD // SUPER
RANGE = 12504
NSPR = 12640
WB_ROWS = 784

_mesh = plsc.VectorSubcoreMesh(core_axis_name="c", subcore_axis_name="s")


def _sc_gather(h, src2d):
    NW = 32
    NST = ERH // SUPER
    TAIL = ERH - NST * SUPER

    @functools.partial(
        pl.kernel,
        out_type=jax.ShapeDtypeStruct((E_HALF, EMB), jnp.float32),
        mesh=_mesh,
        scratch_types=[
            pltpu.VMEM((SUPER, 128), jnp.int32),
            pltpu.VMEM((128, EMB), jnp.float32),
            pltpu.VMEM((128, EMB), jnp.float32),
            pltpu.VMEM((128, EMB), jnp.float32),
            pltpu.VMEM((128, EMB), jnp.float32),
            pltpu.SemaphoreType.DMA,
            pltpu.SemaphoreType.DMA,
            pltpu.SemaphoreType.DMA,
            pltpu.SemaphoreType.DMA,
            pltpu.SemaphoreType.DMA,
            pltpu.SemaphoreType.DMA,
            pltpu.SemaphoreType.DMA,
            pltpu.SemaphoreType.DMA,
        ],
    )
    def k(h_hbm, i_hbm, o_hbm, idx_v, r0, r1, r2, r3,
          g0, g1, g2, g3, w0, w1, w2, w3):
        cid = lax.axis_index("c")
        sid = lax.axis_index("s")
        wid = sid * 2 + cid
        rows = [r0, r1, r2, r3]
        gsem = [g0, g1, g2, g3]
        wsem = [w0, w1, w2, w3]

        @pl.loop(0, (NST + NW - 1) // NW)
        def _(g):
            t = g * NW + wid

            @pl.when(t < NST)
            def _():
                pltpu.sync_copy(i_hbm.at[pl.ds(t * SUPER, SUPER)], idx_v)
                gh = [None] * 4
                wo = [None] * 4
                for j in range(SUPER + 3):
                    if j < SUPER:
                        b = j % 4
                        if wo[b] is not None:
                            wo[b].wait()
                        gh[b] = pltpu.async_copy(
                            h_hbm.at[idx_v.at[j]], rows[b], gsem[b])
                    if j >= 3:
                        p = (j - 3) % 4
                        gh[p].wait()
                        wo[p] = pltpu.async_copy(
                            rows[p],
                            o_hbm.at[pl.ds((t * SUPER + j - 3) * 128, 128)],
                            wsem[p])
                for p in range(4):
                    wo[p].wait()

        @pl.when(wid == 0)
        def _():
            pltpu.sync_copy(
                i_hbm.at[pl.ds(NST * SUPER, TAIL)], idx_v.at[pl.ds(0, TAIL)])
            for j in range(TAIL):
                pltpu.async_copy(h_hbm.at[idx_v.at[j]], r0, g0).wait()
                pltpu.sync_copy(
                    r0, o_hbm.at[pl.ds((NST * SUPER + j) * 128, 128)])

    return k(h, src2d)


def _sc_scatter(m, dst2d):
    @functools.partial(
        pl.kernel,
        out_type=jax.ShapeDtypeStruct((N, EMB), jnp.float32),
        mesh=_mesh,
        scratch_types=[
            pltpu.VMEM_SHARED((NSPR, EMB), jnp.float32),
            pltpu.VMEM((SUPER, 128), jnp.int32),
            pltpu.VMEM((2 * SUPER, 64), jnp.int32),
            pltpu.VMEM((64, EMB), jnp.float32),
            pltpu.VMEM((64, EMB), jnp.float32),
            pltpu.VMEM((64, EMB), jnp.float32),
            pltpu.SemaphoreType.DMA,
            pltpu.SemaphoreType.DMA,
            pltpu.SemaphoreType.DMA,
        ],
    )
    def k(m_hbm, d_hbm, o_hbm, acc, idx_v, idxs_v, mv0, mv1, mv2,
          sl0, sl1, sl2):
        cid = lax.axis_index("c")
        sid = lax.axis_index("s")
        mv = [mv0, mv1, mv2]
        sl = [sl0, sl1, sl2]

        zvec = jnp.zeros((16,), jnp.float32)
        iota16 = lax.iota(jnp.int32, 16)

        def run_pass(r0, nrows):
            @pl.loop(0, 64)
            def _(r):
                @pl.loop(0, EMB // 16)
                def _(c):
                    mv0[r, pl.ds(c * 16, 16)] = zvec

            zbase = sid * (NSPR // NS)

            @pl.loop(0, 12)
            def _(zi):
                pltpu.sync_copy(mv0, acc.at[pl.ds(zbase + zi * 64, 64)])

            pltpu.sync_copy(
                mv0.at[pl.ds(0, (NSPR // NS) - 12 * 64)],
                acc.at[pl.ds(zbase + 12 * 64, (NSPR // NS) - 12 * 64)])
            plsc.subcore_barrier()

            @pl.loop(0, (NSUPH + NS - 1) // NS)
            def _(g):
                t = g * NS + sid

                @pl.when(t < NSUPH)
                def _():
                    pltpu.sync_copy(d_hbm.at[pl.ds(t * SUPER, SUPER)], idx_v)
                    for r in range(SUPER):
                        for c in range(8):
                            u = idx_v[r, pl.ds(c * 16, 16)] - r0
                            oob = (u < 0) | (u >= nrows)
                            pad = (RANGE + ((r * 8 + c) % 8) * 16) + iota16
                            idxs_v[2 * r + c // 4, pl.ds((c % 4) * 16, 16)] \
                                = jnp.where(oob, pad, u)
                    lh = [None, None, None]
                    for p in range(2):
                        lh[p] = pltpu.async_copy(
                            m_hbm.at[pl.ds(t * SUPER * 128 + p * 64, 64)],
                            mv[p], sl[p])
                    for j in range(2 * SUPER):
                        b = j % 3
                        lh[b].wait()
                        if j + 2 < 2 * SUPER:
                            nb = (j + 2) % 3
                            lh[nb] = pltpu.async_copy(
                                m_hbm.at[pl.ds(t * SUPER * 128
                                               + (j + 2) * 64, 64)],
                                mv[nb], sl[nb])
                        pltpu.sync_copy(
                            mv[b], acc.at[idxs_v.at[j]], add=True)

            plsc.subcore_barrier()

            last = nrows - (NS - 1) * WB_ROWS

            @pl.when(sid < NS - 1)
            def _():
                pltpu.sync_copy(
                    acc.at[pl.ds(sid * WB_ROWS, WB_ROWS)],
                    o_hbm.at[pl.ds(r0 + sid * WB_ROWS, WB_ROWS)])

            @pl.when(sid == NS - 1)
            def _():
                pltpu.sync_copy(
                    acc.at[pl.ds((NS - 1) * WB_ROWS, last)],
                    o_hbm.at[pl.ds(r0 + (NS - 1) * WB_ROWS, last)])

            plsc.subcore_barrier()

        @pl.when(cid == 0)
        def _():
            run_pass(0, RANGE)
            run_pass(RANGE, RANGE)

        @pl.when(cid == 1)
        def _():
            run_pass(2 * RANGE, RANGE)
            run_pass(3 * RANGE, N - 3 * RANGE)

    return k(m, dst2d)


def _edge_body(hs_ref, ea_ref, w_ref, b_ref, o_ref):
    e = jnp.dot(ea_ref[...], w_ref[...], preferred_element_type=jnp.float32)
    o_ref[...] = jnp.maximum(hs_ref[...] + e + b_ref[...], 0.0)


def _edge_kernel(hs, edge_attr, eW, eb, half):
    off = half * (E_HALF // EDGE_BLK)
    return pl.pallas_call(
        _edge_body,
        grid=(E_HALF // EDGE_BLK,),
        in_specs=[
            pl.BlockSpec((EDGE_BLK, EMB), lambda i: (i, 0)),
            pl.BlockSpec((EDGE_BLK, 50), lambda i: (i + off, 0)),
            pl.BlockSpec((50, EMB), lambda i: (0, 0)),
            pl.BlockSpec((1, EMB), lambda i: (0, 0)),
        ],
        out_specs=pl.BlockSpec((EDGE_BLK, EMB), lambda i: (i, 0)),
        out_shape=jax.ShapeDtypeStruct((EH_PAD, EMB), jnp.float32),
    )(hs, edge_attr, eW, eb.reshape(1, EMB))


def _node_a_body(h_ref, aA_ref, aB_ref, w1_ref, b1_ref, w2_ref, b2_ref,
                 eps_ref, z2_ref, st_ref, sacc, ssacc):
    i = pl.program_id(0)

    @pl.when(i == 0)
    def _():
        sacc[...] = jnp.zeros_like(sacc)
        ssacc[...] = jnp.zeros_like(ssacc)

    z = (1.0 + eps_ref[0, 0]) * h_ref[...] + aA_ref[...] + aB_ref[...]
    z1 = jnp.maximum(
        jnp.dot(z, w1_ref[...], preferred_element_type=jnp.float32)
        + b1_ref[...], 0.0)
    z2 = jnp.dot(z1, w2_ref[...], preferred_element_type=jnp.float32) \
        + b2_ref[...]
    z2_ref[...] = z2
    sacc[...] += jnp.sum(z2, axis=0, keepdims=True)
    ssacc[...] += jnp.sum(z2 * z2, axis=0, keepdims=True)

    @pl.when(i == pl.num_programs(0) - 1)
    def _():
        st_ref[...] = jnp.concatenate([sacc[...], ssacc[...]], axis=0)


def _node_a(h, aA, aB, w1, b1, w2, b2, eps_l):
    return pl.pallas_call(
        _node_a_body,
        grid=(N // NODE_BLK,),
        in_specs=[
            pl.BlockSpec((NODE_BLK, EMB), lambda i: (i, 0)),
            pl.BlockSpec((NODE_BLK, EMB), lambda i: (i, 0)),
            pl.BlockSpec((NODE_BLK, EMB), lambda i: (i, 0)),
            pl.BlockSpec((EMB, EMB), lambda i: (0, 0)),
            pl.BlockSpec((1, EMB), lambda i: (0, 0)),
            pl.BlockSpec((EMB, EMB), lambda i: (0, 0)),
            pl.BlockSpec((1, EMB), lambda i: (0, 0)),
            pl.BlockSpec((1, 1), lambda i: (0, 0)),
        ],
        out_specs=[
            pl.BlockSpec((NODE_BLK, EMB), lambda i: (i, 0)),
            pl.BlockSpec((2, EMB), lambda i: (0, 0)),
        ],
        out_shape=[
            jax.ShapeDtypeStruct((N, EMB), jnp.float32),
            jax.ShapeDtypeStruct((2, EMB), jnp.float32),
        ],
        scratch_shapes=[
            pltpu.VMEM((1, EMB), jnp.float32),
            pltpu.VMEM((1, EMB), jnp.float32),
        ],
    )(h, aA, aB, w1, b1.reshape(1, EMB), w2, b2.reshape(1, EMB), eps_l)


def _norm_body(z2_ref, st_ref, g_ref, b_ref, o_ref, *, with_relu):
    mu = st_ref[0:1, :] * (1.0 / N)
    var = st_ref[1:2, :] * (1.0 / N) - mu * mu
    inv = lax.rsqrt(var + 1e-5)
    o = (z2_ref[...] - mu) * inv * g_ref[...] + b_ref[...]
    if with_relu:
        o = jnp.maximum(o, 0.0)
    o_ref[...] = o


def _node_b(z2, st, gamma, beta, with_relu):
    return pl.pallas_call(
        functools.partial(_norm_body, with_relu=with_relu),
        grid=(N // NODE_BLK,),
        in_specs=[
            pl.BlockSpec((NODE_BLK, EMB), lambda i: (i, 0)),
            pl.BlockSpec((2, EMB), lambda i: (0, 0)),
            pl.BlockSpec((1, EMB), lambda i: (0, 0)),
            pl.BlockSpec((1, EMB), lambda i: (0, 0)),
        ],
        out_specs=pl.BlockSpec((NODE_BLK, EMB), lambda i: (i, 0)),
        out_shape=jax.ShapeDtypeStruct((N, EMB), jnp.float32),
    )(z2, st, gamma.reshape(1, EMB), beta.reshape(1, EMB))


def _mm_stats_body(h_ref, w_ref, b_ref, t_ref, st_ref, sacc, ssacc):
    i = pl.program_id(0)

    @pl.when(i == 0)
    def _():
        sacc[...] = jnp.zeros_like(sacc)
        ssacc[...] = jnp.zeros_like(ssacc)

    t = jnp.dot(h_ref[...], w_ref[...], preferred_element_type=jnp.float32) \
        + b_ref[...]
    t_ref[...] = t
    sacc[...] += jnp.sum(t, axis=0, keepdims=True)
    ssacc[...] += jnp.sum(t * t, axis=0, keepdims=True)

    @pl.when(i == pl.num_programs(0) - 1)
    def _():
        st_ref[...] = jnp.concatenate([sacc[...], ssacc[...]], axis=0)


def _mm_stats(h, w, b):
    return pl.pallas_call(
        _mm_stats_body,
        grid=(N // NODE_BLK,),
        in_specs=[
            pl.BlockSpec((NODE_BLK, EMB), lambda i: (i, 0)),
            pl.BlockSpec((EMB, EMB), lambda i: (0, 0)),
            pl.BlockSpec((1, EMB), lambda i: (0, 0)),
        ],
        out_specs=[
            pl.BlockSpec((NODE_BLK, EMB), lambda i: (i, 0)),
            pl.BlockSpec((2, EMB), lambda i: (0, 0)),
        ],
        out_shape=[
            jax.ShapeDtypeStruct((N, EMB), jnp.float32),
            jax.ShapeDtypeStruct((2, EMB), jnp.float32),
        ],
        scratch_shapes=[
            pltpu.VMEM((1, EMB), jnp.float32),
            pltpu.VMEM((1, EMB), jnp.float32),
        ],
    )(h, w, b.reshape(1, EMB))


def _gate2_body(t_ref, st_ref, g_ref, be_ref, w2_ref, b2_ref, bat_ref,
                gate_ref, gmax_ref, macc):
    i = pl.program_id(0)

    @pl.when(i == 0)
    def _():
        macc[...] = jnp.full_like(macc, -jnp.inf)

    mu = st_ref[0:1, :] * (1.0 / N)
    var = st_ref[1:2, :] * (1.0 / N) - mu * mu
    inv = lax.rsqrt(var + 1e-5)
    t = jnp.maximum((t_ref[...] - mu) * inv * g_ref[...] + be_ref[...], 0.0)
    gate = lax.dot_general(
        w2_ref[...], t,
        dimension_numbers=(((0,), (1,)), ((), ())),
        preferred_element_type=jnp.float32) + b2_ref[0, 0]
    gate_ref[0, :, :] = gate
    ids = bat_ref[0, :, :]
    onehot = (ids == lax.broadcasted_iota(jnp.int32, (B, NODE_BLK), 0))
    masked = jnp.where(onehot, gate, -jnp.inf)
    macc[...] = jnp.maximum(macc[...], jnp.max(masked, axis=1, keepdims=True))

    @pl.when(i == pl.num_programs(0) - 1)
    def _():
        gmax_ref[...] = jnp.where(
            jnp.isfinite(macc[...]), macc[...], 0.0)


def _gate2(t, st, gamma, beta, w2, b2, batch3d):
    return pl.pallas_call(
        _gate2_body,
        grid=(N // NODE_BLK,),
        in_specs=[
            pl.BlockSpec((NODE_BLK, EMB), lambda i: (i, 0)),
            pl.BlockSpec((2, EMB), lambda i: (0, 0)),
            pl.BlockSpec((1, EMB), lambda i: (0, 0)),
            pl.BlockSpec((1, EMB), lambda i: (0, 0)),
            pl.BlockSpec((EMB, 1), lambda i: (0, 0)),
            pl.BlockSpec((1, 1), lambda i: (0, 0)),
            pl.BlockSpec((1, 1, NODE_BLK), lambda i: (i, 0, 0)),
        ],
        out_specs=[
            pl.BlockSpec((1, 1, NODE_BLK), lambda i: (i, 0, 0)),
            pl.BlockSpec((B, 1), lambda i: (0, 0)),
        ],
        out_shape=[
            jax.ShapeDtypeStruct((N // NODE_BLK, 1, NODE_BLK), jnp.float32),
            jax.ShapeDtypeStruct((B, 1), jnp.float32),
        ],
        scratch_shapes=[pltpu.VMEM((B, 1), jnp.float32)],
    )(t, st, gamma.reshape(1, EMB), beta.reshape(1, EMB), w2,
      b2.reshape(1, 1), batch3d)


def _gate3_body(gate_ref, gmax_ref, bat_ref, ex_ref, den_ref, dacc):
    i = pl.program_id(0)

    @pl.when(i == 0)
    def _():
        dacc[...] = jnp.zeros_like(dacc)

    ids = bat_ref[0, :, :]
    onehot_f = (ids == lax.broadcasted_iota(jnp.int32, (B, NODE_BLK), 0)
                ).astype(jnp.float32)
    gmaxb = lax.dot_general(
        gmax_ref[...], onehot_f,
        dimension_numbers=(((0,), (0,)), ((), ())),
        preferred_element_type=jnp.float32)
    ex = jnp.exp(gate_ref[0, :, :] - gmaxb)
    ex_ref[0, :, :] = ex
    dacc[...] += lax.dot_general(
        onehot_f, ex,
        dimension_numbers=(((1,), (1,)), ((), ())),
        preferred_element_type=jnp.float32)

    @pl.when(i == pl.num_programs(0) - 1)
    def _():
        den_ref[...] = dacc[...]


def _gate3(gate_r, gmax, batch3d):
    return pl.pallas_call(
        _gate3_body,
        grid=(N // NODE_BLK,),
        in_specs=[
            pl.BlockSpec((1, 1, NODE_BLK), lambda i: (i, 0, 0)),
            pl.BlockSpec((B, 1), lambda i: (0, 0)),
            pl.BlockSpec((1, 1, NODE_BLK), lambda i: (i, 0, 0)),
        ],
        out_specs=[
            pl.BlockSpec((1, 1, NODE_BLK), lambda i: (i, 0, 0)),
            pl.BlockSpec((B, 1), lambda i: (0, 0)),
        ],
        out_shape=[
            jax.ShapeDtypeStruct((N // NODE_BLK, 1, NODE_BLK), jnp.float32),
            jax.ShapeDtypeStruct((B, 1), jnp.float32),
        ],
        scratch_shapes=[pltpu.VMEM((B, 1), jnp.float32)],
    )(gate_r, gmax, batch3d)


def _gate4_body(ex_ref, den_ref, h_ref, bat_ref, hg_ref, hacc):
    i = pl.program_id(0)

    @pl.when(i == 0)
    def _():
        hacc[...] = jnp.zeros_like(hacc)

    ids = bat_ref[0, :, :]
    onehot_f = (ids == lax.broadcasted_iota(jnp.int32, (B, NODE_BLK), 0)
                ).astype(jnp.float32)
    denb = lax.dot_general(
        den_ref[...], onehot_f,
        dimension_numbers=(((0,), (0,)), ((), ())),
        preferred_element_type=jnp.float32)
    alpha = ex_ref[0, :, :] / denb
    ow = onehot_f * alpha
    hacc[...] += jnp.dot(ow, h_ref[...],
                         preferred_element_type=jnp.float32)

    @pl.when(i == pl.num_programs(0) - 1)
    def _():
        hg_ref[...] = hacc[...]


def _gate4(ex_r, den, h, batch3d):
    return pl.pallas_call(
        _gate4_body,
        grid=(N // NODE_BLK,),
        in_specs=[
            pl.BlockSpec((1, 1, NODE_BLK), lambda i: (i, 0, 0)),
            pl.BlockSpec((B, 1), lambda i: (0, 0)),
            pl.BlockSpec((NODE_BLK, EMB), lambda i: (i, 0)),
            pl.BlockSpec((1, 1, NODE_BLK), lambda i: (i, 0, 0)),
        ],
        out_specs=pl.BlockSpec((B, EMB), lambda i: (0, 0)),
        out_shape=jax.ShapeDtypeStruct((B, EMB), jnp.float32),
        scratch_shapes=[pltpu.VMEM((B, EMB), jnp.float32)],
    )(ex_r, den, h, batch3d)


def _atom_body(x_ref, w_ref, b_ref, o_ref):
    o_ref[...] = jnp.dot(x_ref[...], w_ref[...],
                         preferred_element_type=jnp.float32) + b_ref[...]


def _atom(x, w, b):
    return pl.pallas_call(
        _atom_body,
        grid=(N // NODE_BLK,),
        in_specs=[
            pl.BlockSpec((NODE_BLK, 92), lambda i: (i, 0)),
            pl.BlockSpec((92, EMB), lambda i: (0, 0)),
            pl.BlockSpec((1, EMB), lambda i: (0, 0)),
        ],
        out_specs=pl.BlockSpec((NODE_BLK, EMB), lambda i: (i, 0)),
        out_shape=jax.ShapeDtypeStruct((N, EMB), jnp.float32),
    )(x, w, b.reshape(1, EMB))


def _head_body(hg_ref, g_ref, w1_ref, b1_ref, w2_ref, b2_ref, w3_ref,
               b3_ref, o_ref):
    hcat = jnp.concatenate([hg_ref[...], g_ref[...]], axis=1)
    o = jnp.maximum(jnp.dot(hcat, w1_ref[...],
                            preferred_element_type=jnp.float32)
                    + b1_ref[...], 0.0)
    o = jnp.maximum(jnp.dot(o, w2_ref[...],
                            preferred_element_type=jnp.float32)
                    + b2_ref[...], 0.0)
    o_ref[...] = jnp.dot(o, w3_ref[...],
                         preferred_element_type=jnp.float32) + b3_ref[...]


def _head(hg, g, w1, b1, w2, b2, w3, b3):
    H0 = EMB + 10
    return pl.pallas_call(
        _head_body,
        in_specs=[
            pl.BlockSpec((B, EMB), lambda: (0, 0)),
            pl.BlockSpec((B, 10), lambda: (0, 0)),
            pl.BlockSpec((H0, 2 * H0), lambda: (0, 0)),
            pl.BlockSpec((1, 2 * H0), lambda: (0, 0)),
            pl.BlockSpec((2 * H0, H0), lambda: (0, 0)),
            pl.BlockSpec((1, H0), lambda: (0, 0)),
            pl.BlockSpec((H0, 1), lambda: (0, 0)),
            pl.BlockSpec((1, 1), lambda: (0, 0)),
        ],
        out_specs=pl.BlockSpec((B, 1), lambda: (0, 0)),
        out_shape=jax.ShapeDtypeStruct((B, 1), jnp.float32),
    )(hg, g, w1, b1.reshape(1, 2 * H0), w2, b2.reshape(1, H0), w3,
      b3.reshape(1, 1))


def kernel(x, edge_index, edge_attr, batch, ptr, g, atom_W, atom_b, edge_W,
           edge_b, W1, b1, W2, b2, eps, bn_g, bn_b, gate_W1, gate_b1,
           gate_bn_g, gate_bn_b, gate_W2, gate_b2, h_W1, h_b1, h_W2, h_b2,
           h_W3, h_b3):
    src = edge_index[1]
    dst = edge_index[0]
    srcA = src[:E_HALF].reshape(ERH, 128)
    srcB = src[E_HALF:].reshape(ERH, 128)
    pad_idx = jnp.full((EH_PAD - E_HALF,), N, jnp.int32)
    dstA = jnp.concatenate([dst[:E_HALF], pad_idx]).reshape(ERH_PAD, 128)
    dstB = jnp.concatenate([dst[E_HALF:], pad_idx]).reshape(ERH_PAD, 128)
    batch3d = batch.reshape(N // NODE_BLK, 1, NODE_BLK)

    h = _atom(x, atom_W, atom_b)
    for l in range(4):
        hsA = _sc_gather(h, srcA)
        mA = _edge_kernel(hsA, edge_attr, edge_W[l], edge_b[l], 0)
        hsB = _sc_gather(h, srcB)
        mB = _edge_kernel(hsB, edge_attr, edge_W[l], edge_b[l], 1)
        aA = _sc_scatter(mA, dstA)
        aB = _sc_scatter(mB, dstB)
        z2, st = _node_a(h, aA, aB, W1[l], b1[l], W2[l], b2[l],
                         eps[l].reshape(1, 1))
        h = _node_b(z2, st, bn_g[l], bn_b[l], with_relu=(l < 3))

    t, tst = _mm_stats(h, gate_W1, gate_b1)
    gate_r, gmax = _gate2(t, tst, gate_bn_g, gate_bn_b, gate_W2, gate_b2,
                          batch3d)
    ex_r, den = _gate3(gate_r, gmax, batch3d)
    hg = _gate4(ex_r, den, h, batch3d)
    return _head(hg, g, h_W1, h_b1, h_W2, h_b2, h_W3, h_b3)


# speedup vs baseline: 2.1939x; 1.0014x over previous
import functools

import jax
import jax.numpy as jnp
from jax import lax
from jax.experimental import pallas as pl
from jax.experimental.pallas import tpu as pltpu
from jax.experimental.pallas import tpu_sc as plsc

N = 50000
E = 800000
B = 128
EMB = 128
NODE_BLK = 2000
E_HALF = E // 2
EH_PAD = 400384
ERH = E_HALF // 128
ERH_PAD = EH_PAD // 128
EDGE_BLK = 2000
NS = 16
SUPER = 8
NSUPH = ERH_PAD // SUPER
RANGE = 12504
NSPR = 12640
WB_ROWS = 784

_mesh = plsc.VectorSubcoreMesh(core_axis_name="c", subcore_axis_name="s")


def _sc_gather(h, src2d):
    NW = 32
    NST = ERH // SUPER
    TAIL = ERH - NST * SUPER

    @functools.partial(
        pl.kernel,
        out_type=jax.ShapeDtypeStruct((E_HALF, EMB), jnp.float32),
        mesh=_mesh,
        scratch_types=[
            pltpu.VMEM((SUPER, 128), jnp.int32),
            pltpu.VMEM((128, EMB), jnp.float32),
            pltpu.VMEM((128, EMB), jnp.float32),
            pltpu.VMEM((128, EMB), jnp.float32),
            pltpu.VMEM((128, EMB), jnp.float32),
            pltpu.VMEM((128, EMB), jnp.float32),
            pltpu.VMEM((128, EMB), jnp.float32),
            pltpu.SemaphoreType.DMA,
            pltpu.SemaphoreType.DMA,
            pltpu.SemaphoreType.DMA,
            pltpu.SemaphoreType.DMA,
            pltpu.SemaphoreType.DMA,
            pltpu.SemaphoreType.DMA,
            pltpu.SemaphoreType.DMA,
            pltpu.SemaphoreType.DMA,
            pltpu.SemaphoreType.DMA,
            pltpu.SemaphoreType.DMA,
            pltpu.SemaphoreType.DMA,
            pltpu.SemaphoreType.DMA,
        ],
    )
    def k(h_hbm, i_hbm, o_hbm, idx_v, r0, r1, r2, r3, r4, r5,
          g0, g1, g2, g3, g4, g5, w0, w1, w2, w3, w4, w5):
        cid = lax.axis_index("c")
        sid = lax.axis_index("s")
        wid = sid * 2 + cid
        rows = [r0, r1, r2, r3, r4, r5]
        gsem = [g0, g1, g2, g3, g4, g5]
        wsem = [w0, w1, w2, w3, w4, w5]

        @pl.loop(0, (NST + NW - 1) // NW)
        def _(g):
            t = g * NW + wid

            @pl.when(t < NST)
            def _():
                pltpu.sync_copy(i_hbm.at[pl.ds(t * SUPER, SUPER)], idx_v)
                gh = [None] * 6
                wo = [None] * 6
                for j in range(SUPER + 4):
                    if j < SUPER:
                        b = j % 6
                        if wo[b] is not None:
                            wo[b].wait()
                        gh[b] = pltpu.async_copy(
                            h_hbm.at[idx_v.at[j]], rows[b], gsem[b])
                    if j >= 4:
                        p = (j - 4) % 6
                        gh[p].wait()
                        wo[p] = pltpu.async_copy(
                            rows[p],
                            o_hbm.at[pl.ds((t * SUPER + j - 4) * 128, 128)],
                            wsem[p])
                for p in range(6):
                    if wo[p] is not None:
                        wo[p].wait()

        @pl.when(wid == 0)
        def _():
            pltpu.sync_copy(
                i_hbm.at[pl.ds(NST * SUPER, TAIL)], idx_v.at[pl.ds(0, TAIL)])
            for j in range(TAIL):
                pltpu.async_copy(h_hbm.at[idx_v.at[j]], r0, g0).wait()
                pltpu.sync_copy(
                    r0, o_hbm.at[pl.ds((NST * SUPER + j) * 128, 128)])

    return k(h, src2d)


def _sc_scatter(m, dst2d):
    @functools.partial(
        pl.kernel,
        out_type=jax.ShapeDtypeStruct((N, EMB), jnp.float32),
        mesh=_mesh,
        scratch_types=[
            pltpu.VMEM_SHARED((NSPR, EMB), jnp.float32),
            pltpu.VMEM((SUPER, 128), jnp.int32),
            pltpu.VMEM((2 * SUPER, 64), jnp.int32),
            pltpu.VMEM((64, EMB), jnp.float32),
            pltpu.VMEM((64, EMB), jnp.float32),
            pltpu.VMEM((64, EMB), jnp.float32),
            pltpu.SemaphoreType.DMA,
            pltpu.SemaphoreType.DMA,
            pltpu.SemaphoreType.DMA,
        ],
    )
    def k(m_hbm, d_hbm, o_hbm, acc, idx_v, idxs_v, mv0, mv1, mv2,
          sl0, sl1, sl2):
        cid = lax.axis_index("c")
        sid = lax.axis_index("s")
        mv = [mv0, mv1, mv2]
        sl = [sl0, sl1, sl2]

        zvec = jnp.zeros((16,), jnp.float32)
        iota16 = lax.iota(jnp.int32, 16)

        def run_pass(r0, nrows):
            @pl.loop(0, 64)
            def _(r):
                @pl.loop(0, EMB // 16)
                def _(c):
                    mv0[r, pl.ds(c * 16, 16)] = zvec

            zbase = sid * (NSPR // NS)

            @pl.loop(0, 12)
            def _(zi):
                pltpu.sync_copy(mv0, acc.at[pl.ds(zbase + zi * 64, 64)])

            pltpu.sync_copy(
                mv0.at[pl.ds(0, (NSPR // NS) - 12 * 64)],
                acc.at[pl.ds(zbase + 12 * 64, (NSPR // NS) - 12 * 64)])
            plsc.subcore_barrier()

            @pl.loop(0, (NSUPH + NS - 1) // NS)
            def _(g):
                t = g * NS + sid

                @pl.when(t < NSUPH)
                def _():
                    pltpu.sync_copy(d_hbm.at[pl.ds(t * SUPER, SUPER)], idx_v)
                    for r in range(SUPER):
                        for c in range(8):
                            u = idx_v[r, pl.ds(c * 16, 16)] - r0
                            oob = (u < 0) | (u >= nrows)
                            pad = (RANGE + ((r * 8 + c) % 8) * 16) + iota16
                            idxs_v[2 * r + c // 4, pl.ds((c % 4) * 16, 16)] \
                                = jnp.where(oob, pad, u)
                    lh = [None, None, None]
                    for p in range(2):
                        lh[p] = pltpu.async_copy(
                            m_hbm.at[pl.ds(t * SUPER * 128 + p * 64, 64)],
                            mv[p], sl[p])
                    for j in range(2 * SUPER):
                        b = j % 3
                        lh[b].wait()
                        if j + 2 < 2 * SUPER:
                            nb = (j + 2) % 3
                            lh[nb] = pltpu.async_copy(
                                m_hbm.at[pl.ds(t * SUPER * 128
                                               + (j + 2) * 64, 64)],
                                mv[nb], sl[nb])
                        pltpu.sync_copy(
                            mv[b], acc.at[idxs_v.at[j]], add=True)

            plsc.subcore_barrier()

            last = nrows - (NS - 1) * WB_ROWS

            @pl.when(sid < NS - 1)
            def _():
                pltpu.sync_copy(
                    acc.at[pl.ds(sid * WB_ROWS, WB_ROWS)],
                    o_hbm.at[pl.ds(r0 + sid * WB_ROWS, WB_ROWS)])

            @pl.when(sid == NS - 1)
            def _():
                pltpu.sync_copy(
                    acc.at[pl.ds((NS - 1) * WB_ROWS, last)],
                    o_hbm.at[pl.ds(r0 + (NS - 1) * WB_ROWS, last)])

            plsc.subcore_barrier()

        @pl.when(cid == 0)
        def _():
            run_pass(0, RANGE)
            run_pass(RANGE, RANGE)

        @pl.when(cid == 1)
        def _():
            run_pass(2 * RANGE, RANGE)
            run_pass(3 * RANGE, N - 3 * RANGE)

    return k(m, dst2d)


def _edge_body(hs_ref, ea_ref, w_ref, b_ref, o_ref):
    e = jnp.dot(ea_ref[...], w_ref[...], preferred_element_type=jnp.float32)
    o_ref[...] = jnp.maximum(hs_ref[...] + e + b_ref[...], 0.0)


def _edge_kernel(hs, edge_attr, eW, eb, half):
    off = half * (E_HALF // EDGE_BLK)
    return pl.pallas_call(
        _edge_body,
        grid=(E_HALF // EDGE_BLK,),
        in_specs=[
            pl.BlockSpec((EDGE_BLK, EMB), lambda i: (i, 0)),
            pl.BlockSpec((EDGE_BLK, 50), lambda i: (i + off, 0)),
            pl.BlockSpec((50, EMB), lambda i: (0, 0)),
            pl.BlockSpec((1, EMB), lambda i: (0, 0)),
        ],
        out_specs=pl.BlockSpec((EDGE_BLK, EMB), lambda i: (i, 0)),
        out_shape=jax.ShapeDtypeStruct((EH_PAD, EMB), jnp.float32),
    )(hs, edge_attr, eW, eb.reshape(1, EMB))


def _node_a_body(h_ref, aA_ref, aB_ref, w1_ref, b1_ref, w2_ref, b2_ref,
                 eps_ref, z2_ref, st_ref, sacc, ssacc):
    i = pl.program_id(0)

    @pl.when(i == 0)
    def _():
        sacc[...] = jnp.zeros_like(sacc)
        ssacc[...] = jnp.zeros_like(ssacc)

    z = (1.0 + eps_ref[0, 0]) * h_ref[...] + aA_ref[...] + aB_ref[...]
    z1 = jnp.maximum(
        jnp.dot(z, w1_ref[...], preferred_element_type=jnp.float32)
        + b1_ref[...], 0.0)
    z2 = jnp.dot(z1, w2_ref[...], preferred_element_type=jnp.float32) \
        + b2_ref[...]
    z2_ref[...] = z2
    sacc[...] += jnp.sum(z2, axis=0, keepdims=True)
    ssacc[...] += jnp.sum(z2 * z2, axis=0, keepdims=True)

    @pl.when(i == pl.num_programs(0) - 1)
    def _():
        st_ref[...] = jnp.concatenate([sacc[...], ssacc[...]], axis=0)


def _node_a(h, aA, aB, w1, b1, w2, b2, eps_l):
    return pl.pallas_call(
        _node_a_body,
        grid=(N // NODE_BLK,),
        in_specs=[
            pl.BlockSpec((NODE_BLK, EMB), lambda i: (i, 0)),
            pl.BlockSpec((NODE_BLK, EMB), lambda i: (i, 0)),
            pl.BlockSpec((NODE_BLK, EMB), lambda i: (i, 0)),
            pl.BlockSpec((EMB, EMB), lambda i: (0, 0)),
            pl.BlockSpec((1, EMB), lambda i: (0, 0)),
            pl.BlockSpec((EMB, EMB), lambda i: (0, 0)),
            pl.BlockSpec((1, EMB), lambda i: (0, 0)),
            pl.BlockSpec((1, 1), lambda i: (0, 0)),
        ],
        out_specs=[
            pl.BlockSpec((NODE_BLK, EMB), lambda i: (i, 0)),
            pl.BlockSpec((2, EMB), lambda i: (0, 0)),
        ],
        out_shape=[
            jax.ShapeDtypeStruct((N, EMB), jnp.float32),
            jax.ShapeDtypeStruct((2, EMB), jnp.float32),
        ],
        scratch_shapes=[
            pltpu.VMEM((1, EMB), jnp.float32),
            pltpu.VMEM((1, EMB), jnp.float32),
        ],
    )(h, aA, aB, w1, b1.reshape(1, EMB), w2, b2.reshape(1, EMB), eps_l)


def _norm_body(z2_ref, st_ref, g_ref, b_ref, o_ref, *, with_relu):
    mu = st_ref[0:1, :] * (1.0 / N)
    var = st_ref[1:2, :] * (1.0 / N) - mu * mu
    inv = lax.rsqrt(var + 1e-5)
    o = (z2_ref[...] - mu) * inv * g_ref[...] + b_ref[...]
    if with_relu:
        o = jnp.maximum(o, 0.0)
    o_ref[...] = o


def _node_b(z2, st, gamma, beta, with_relu):
    return pl.pallas_call(
        functools.partial(_norm_body, with_relu=with_relu),
        grid=(N // NODE_BLK,),
        in_specs=[
            pl.BlockSpec((NODE_BLK, EMB), lambda i: (i, 0)),
            pl.BlockSpec((2, EMB), lambda i: (0, 0)),
            pl.BlockSpec((1, EMB), lambda i: (0, 0)),
            pl.BlockSpec((1, EMB), lambda i: (0, 0)),
        ],
        out_specs=pl.BlockSpec((NODE_BLK, EMB), lambda i: (i, 0)),
        out_shape=jax.ShapeDtypeStruct((N, EMB), jnp.float32),
    )(z2, st, gamma.reshape(1, EMB), beta.reshape(1, EMB))


def _mm_stats_body(h_ref, w_ref, b_ref, t_ref, st_ref, sacc, ssacc):
    i = pl.program_id(0)

    @pl.when(i == 0)
    def _():
        sacc[...] = jnp.zeros_like(sacc)
        ssacc[...] = jnp.zeros_like(ssacc)

    t = jnp.dot(h_ref[...], w_ref[...], preferred_element_type=jnp.float32) \
        + b_ref[...]
    t_ref[...] = t
    sacc[...] += jnp.sum(t, axis=0, keepdims=True)
    ssacc[...] += jnp.sum(t * t, axis=0, keepdims=True)

    @pl.when(i == pl.num_programs(0) - 1)
    def _():
        st_ref[...] = jnp.concatenate([sacc[...], ssacc[...]], axis=0)


def _mm_stats(h, w, b):
    return pl.pallas_call(
        _mm_stats_body,
        grid=(N // NODE_BLK,),
        in_specs=[
            pl.BlockSpec((NODE_BLK, EMB), lambda i: (i, 0)),
            pl.BlockSpec((EMB, EMB), lambda i: (0, 0)),
            pl.BlockSpec((1, EMB), lambda i: (0, 0)),
        ],
        out_specs=[
            pl.BlockSpec((NODE_BLK, EMB), lambda i: (i, 0)),
            pl.BlockSpec((2, EMB), lambda i: (0, 0)),
        ],
        out_shape=[
            jax.ShapeDtypeStruct((N, EMB), jnp.float32),
            jax.ShapeDtypeStruct((2, EMB), jnp.float32),
        ],
        scratch_shapes=[
            pltpu.VMEM((1, EMB), jnp.float32),
            pltpu.VMEM((1, EMB), jnp.float32),
        ],
    )(h, w, b.reshape(1, EMB))


def _gate2_body(t_ref, st_ref, g_ref, be_ref, w2_ref, b2_ref, bat_ref,
                gate_ref, gmax_ref, macc):
    i = pl.program_id(0)

    @pl.when(i == 0)
    def _():
        macc[...] = jnp.full_like(macc, -jnp.inf)

    mu = st_ref[0:1, :] * (1.0 / N)
    var = st_ref[1:2, :] * (1.0 / N) - mu * mu
    inv = lax.rsqrt(var + 1e-5)
    t = jnp.maximum((t_ref[...] - mu) * inv * g_ref[...] + be_ref[...], 0.0)
    gate = lax.dot_general(
        w2_ref[...], t,
        dimension_numbers=(((0,), (1,)), ((), ())),
        preferred_element_type=jnp.float32) + b2_ref[0, 0]
    gate_ref[0, :, :] = gate
    ids = bat_ref[0, :, :]
    onehot = (ids == lax.broadcasted_iota(jnp.int32, (B, NODE_BLK), 0))
    masked = jnp.where(onehot, gate, -jnp.inf)
    macc[...] = jnp.maximum(macc[...], jnp.max(masked, axis=1, keepdims=True))

    @pl.when(i == pl.num_programs(0) - 1)
    def _():
        gmax_ref[...] = jnp.where(
            jnp.isfinite(macc[...]), macc[...], 0.0)


def _gate2(t, st, gamma, beta, w2, b2, batch3d):
    return pl.pallas_call(
        _gate2_body,
        grid=(N // NODE_BLK,),
        in_specs=[
            pl.BlockSpec((NODE_BLK, EMB), lambda i: (i, 0)),
            pl.BlockSpec((2, EMB), lambda i: (0, 0)),
            pl.BlockSpec((1, EMB), lambda i: (0, 0)),
            pl.BlockSpec((1, EMB), lambda i: (0, 0)),
            pl.BlockSpec((EMB, 1), lambda i: (0, 0)),
            pl.BlockSpec((1, 1), lambda i: (0, 0)),
            pl.BlockSpec((1, 1, NODE_BLK), lambda i: (i, 0, 0)),
        ],
        out_specs=[
            pl.BlockSpec((1, 1, NODE_BLK), lambda i: (i, 0, 0)),
            pl.BlockSpec((B, 1), lambda i: (0, 0)),
        ],
        out_shape=[
            jax.ShapeDtypeStruct((N // NODE_BLK, 1, NODE_BLK), jnp.float32),
            jax.ShapeDtypeStruct((B, 1), jnp.float32),
        ],
        scratch_shapes=[pltpu.VMEM((B, 1), jnp.float32)],
    )(t, st, gamma.reshape(1, EMB), beta.reshape(1, EMB), w2,
      b2.reshape(1, 1), batch3d)


def _gate3_body(gate_ref, gmax_ref, bat_ref, ex_ref, den_ref, dacc):
    i = pl.program_id(0)

    @pl.when(i == 0)
    def _():
        dacc[...] = jnp.zeros_like(dacc)

    ids = bat_ref[0, :, :]
    onehot_f = (ids == lax.broadcasted_iota(jnp.int32, (B, NODE_BLK), 0)
                ).astype(jnp.float32)
    gmaxb = lax.dot_general(
        gmax_ref[...], onehot_f,
        dimension_numbers=(((0,), (0,)), ((), ())),
        preferred_element_type=jnp.float32)
    ex = jnp.exp(gate_ref[0, :, :] - gmaxb)
    ex_ref[0, :, :] = ex
    dacc[...] += lax.dot_general(
        onehot_f, ex,
        dimension_numbers=(((1,), (1,)), ((), ())),
        preferred_element_type=jnp.float32)

    @pl.when(i == pl.num_programs(0) - 1)
    def _():
        den_ref[...] = dacc[...]


def _gate3(gate_r, gmax, batch3d):
    return pl.pallas_call(
        _gate3_body,
        grid=(N // NODE_BLK,),
        in_specs=[
            pl.BlockSpec((1, 1, NODE_BLK), lambda i: (i, 0, 0)),
            pl.BlockSpec((B, 1), lambda i: (0, 0)),
            pl.BlockSpec((1, 1, NODE_BLK), lambda i: (i, 0, 0)),
        ],
        out_specs=[
            pl.BlockSpec((1, 1, NODE_BLK), lambda i: (i, 0, 0)),
            pl.BlockSpec((B, 1), lambda i: (0, 0)),
        ],
        out_shape=[
            jax.ShapeDtypeStruct((N // NODE_BLK, 1, NODE_BLK), jnp.float32),
            jax.ShapeDtypeStruct((B, 1), jnp.float32),
        ],
        scratch_shapes=[pltpu.VMEM((B, 1), jnp.float32)],
    )(gate_r, gmax, batch3d)


def _gate4_body(ex_ref, den_ref, h_ref, bat_ref, hg_ref, hacc):
    i = pl.program_id(0)

    @pl.when(i == 0)
    def _():
        hacc[...] = jnp.zeros_like(hacc)

    ids = bat_ref[0, :, :]
    onehot_f = (ids == lax.broadcasted_iota(jnp.int32, (B, NODE_BLK), 0)
                ).astype(jnp.float32)
    denb = lax.dot_general(
        den_ref[...], onehot_f,
        dimension_numbers=(((0,), (0,)), ((), ())),
        preferred_element_type=jnp.float32)
    alpha = ex_ref[0, :, :] / denb
    ow = onehot_f * alpha
    hacc[...] += jnp.dot(ow, h_ref[...],
                         preferred_element_type=jnp.float32)

    @pl.when(i == pl.num_programs(0) - 1)
    def _():
        hg_ref[...] = hacc[...]


def _gate4(ex_r, den, h, batch3d):
    return pl.pallas_call(
        _gate4_body,
        grid=(N // NODE_BLK,),
        in_specs=[
            pl.BlockSpec((1, 1, NODE_BLK), lambda i: (i, 0, 0)),
            pl.BlockSpec((B, 1), lambda i: (0, 0)),
            pl.BlockSpec((NODE_BLK, EMB), lambda i: (i, 0)),
            pl.BlockSpec((1, 1, NODE_BLK), lambda i: (i, 0, 0)),
        ],
        out_specs=pl.BlockSpec((B, EMB), lambda i: (0, 0)),
        out_shape=jax.ShapeDtypeStruct((B, EMB), jnp.float32),
        scratch_shapes=[pltpu.VMEM((B, EMB), jnp.float32)],
    )(ex_r, den, h, batch3d)


def _atom_body(x_ref, w_ref, b_ref, o_ref):
    o_ref[...] = jnp.dot(x_ref[...], w_ref[...],
                         preferred_element_type=jnp.float32) + b_ref[...]


def _atom(x, w, b):
    return pl.pallas_call(
        _atom_body,
        grid=(N // NODE_BLK,),
        in_specs=[
            pl.BlockSpec((NODE_BLK, 92), lambda i: (i, 0)),
            pl.BlockSpec((92, EMB), lambda i: (0, 0)),
            pl.BlockSpec((1, EMB), lambda i: (0, 0)),
        ],
        out_specs=pl.BlockSpec((NODE_BLK, EMB), lambda i: (i, 0)),
        out_shape=jax.ShapeDtypeStruct((N, EMB), jnp.float32),
    )(x, w, b.reshape(1, EMB))


def _head_body(hg_ref, g_ref, w1_ref, b1_ref, w2_ref, b2_ref, w3_ref,
               b3_ref, o_ref):
    hcat = jnp.concatenate([hg_ref[...], g_ref[...]], axis=1)
    o = jnp.maximum(jnp.dot(hcat, w1_ref[...],
                            preferred_element_type=jnp.float32)
                    + b1_ref[...], 0.0)
    o = jnp.maximum(jnp.dot(o, w2_ref[...],
                            preferred_element_type=jnp.float32)
                    + b2_ref[...], 0.0)
    o_ref[...] = jnp.dot(o, w3_ref[...],
                         preferred_element_type=jnp.float32) + b3_ref[...]


def _head(hg, g, w1, b1, w2, b2, w3, b3):
    H0 = EMB + 10
    return pl.pallas_call(
        _head_body,
        in_specs=[
            pl.BlockSpec((B, EMB), lambda: (0, 0)),
            pl.BlockSpec((B, 10), lambda: (0, 0)),
            pl.BlockSpec((H0, 2 * H0), lambda: (0, 0)),
            pl.BlockSpec((1, 2 * H0), lambda: (0, 0)),
            pl.BlockSpec((2 * H0, H0), lambda: (0, 0)),
            pl.BlockSpec((1, H0), lambda: (0, 0)),
            pl.BlockSpec((H0, 1), lambda: (0, 0)),
            pl.BlockSpec((1, 1), lambda: (0, 0)),
        ],
        out_specs=pl.BlockSpec((B, 1), lambda: (0, 0)),
        out_shape=jax.ShapeDtypeStruct((B, 1), jnp.float32),
    )(hg, g, w1, b1.reshape(1, 2 * H0), w2, b2.reshape(1, H0), w3,
      b3.reshape(1, 1))


def kernel(x, edge_index, edge_attr, batch, ptr, g, atom_W, atom_b, edge_W,
           edge_b, W1, b1, W2, b2, eps, bn_g, bn_b, gate_W1, gate_b1,
           gate_bn_g, gate_bn_b, gate_W2, gate_b2, h_W1, h_b1, h_W2, h_b2,
           h_W3, h_b3):
    src = edge_index[1]
    dst = edge_index[0]
    srcA = src[:E_HALF].reshape(ERH, 128)
    srcB = src[E_HALF:].reshape(ERH, 128)
    pad_idx = jnp.full((EH_PAD - E_HALF,), N, jnp.int32)
    dstA = jnp.concatenate([dst[:E_HALF], pad_idx]).reshape(ERH_PAD, 128)
    dstB = jnp.concatenate([dst[E_HALF:], pad_idx]).reshape(ERH_PAD, 128)
    batch3d = batch.reshape(N // NODE_BLK, 1, NODE_BLK)

    h = _atom(x, atom_W, atom_b)
    for l in range(4):
        hsA = _sc_gather(h, srcA)
        mA = _edge_kernel(hsA, edge_attr, edge_W[l], edge_b[l], 0)
        hsB = _sc_gather(h, srcB)
        mB = _edge_kernel(hsB, edge_attr, edge_W[l], edge_b[l], 1)
        aA = _sc_scatter(mA, dstA)
        aB = _sc_scatter(mB, dstB)
        z2, st = _node_a(h, aA, aB, W1[l], b1[l], W2[l], b2[l],
                         eps[l].reshape(1, 1))
        h = _node_b(z2, st, bn_g[l], bn_b[l], with_relu=(l < 3))

    t, tst = _mm_stats(h, gate_W1, gate_b1)
    gate_r, gmax = _gate2(t, tst, gate_bn_g, gate_bn_b, gate_W2, gate_b2,
                          batch3d)
    ex_r, den = _gate3(gate_r, gmax, batch3d)
    hg = _gate4(ex_r, den, h, batch3d)
    return _head(hg, g, h_W1, h_b1, h_W2, h_b2, h_W3, h_b3)
